# Initial kernel scaffold; baseline (speedup 1.0000x reference)
import jax
import jax.numpy as jnp
from jax import lax
from jax.experimental import pallas as pl
from jax.experimental.pallas import tpu as pltpu
from jax.experimental.pallas import tpu_sc as plsc

N = 10000
E = 320000
B = 16
F_IN = 4
H = 128
NUM_CLASSES = 8

NP = 10240
NC = 2
NS = 16
NW = NC * NS
CHUNK = 40
NCH = E // CHUNK
NCH_W = NCH // NW
ROWS_PT = NP // NS
ZR = 64

RB = 512
GRID = NP // RB


def _make_seg_sum(compute_deg):
  mesh = plsc.VectorSubcoreMesh(core_axis_name="c", subcore_axis_name="s")
  out_type = [jax.ShapeDtypeStruct((NC, NP, H), jnp.float32)]
  if compute_deg:
    out_type.append(jax.ShapeDtypeStruct((NC, NP, 16), jnp.float32))

  scratch = [
      pltpu.VMEM((NCH_W, CHUNK), jnp.int32),
      pltpu.VMEM((NCH_W, CHUNK), jnp.int32),
      pltpu.VMEM((CHUNK, H), jnp.float32),
      pltpu.VMEM((ZR, H), jnp.float32),
      pltpu.VMEM_SHARED((NP, H), jnp.float32),
      pltpu.SemaphoreType.DMA,
  ]
  if compute_deg:
    scratch += [
        pltpu.VMEM((CHUNK, 16), jnp.float32),
        pltpu.VMEM((ZR, 16), jnp.float32),
        pltpu.VMEM_SHARED((NP, 16), jnp.float32),
    ]

  def body(h_hbm, src_hbm, dst_hbm, *rest):
    if compute_deg:
      (out_hbm, deg_hbm, src_v, dst_v, rows_v, zbuf, acc_sh, sem,
       ones_v, zdeg, deg_sh) = rest
    else:
      out_hbm, src_v, dst_v, rows_v, zbuf, acc_sh, sem = rest

    cid = lax.axis_index("c")
    sid = lax.axis_index("s")
    wid = sid * NC + cid
    row0 = sid * ROWS_PT

    @pl.loop(0, ZR)
    def _z(i):
      for c in range(H // 16):
        zbuf[i, pl.ds(c * 16, 16)] = jnp.zeros((16,), jnp.float32)

    @pl.loop(0, ROWS_PT // ZR)
    def _za(i):
      pltpu.sync_copy(zbuf, acc_sh.at[pl.ds(row0 + i * ZR, ZR)])

    if compute_deg:
      @pl.loop(0, ZR)
      def _zd(i):
        zdeg[i, :] = jnp.zeros((16,), jnp.float32)

      @pl.loop(0, ROWS_PT // ZR)
      def _zda(i):
        pltpu.sync_copy(zdeg, deg_sh.at[pl.ds(row0 + i * ZR, ZR)])

      @pl.loop(0, CHUNK)
      def _o(i):
        ones_v[i, :] = jnp.ones((16,), jnp.float32)

    pltpu.sync_copy(src_hbm.at[pl.ds(wid * NCH_W, NCH_W)], src_v)
    pltpu.sync_copy(dst_hbm.at[pl.ds(wid * NCH_W, NCH_W)], dst_v)

    plsc.subcore_barrier()

    @pl.loop(0, NCH_W)
    def _edges(j):
      pltpu.async_copy(h_hbm.at[src_v.at[j]], rows_v, sem).wait()
      pltpu.sync_copy(rows_v, acc_sh.at[dst_v.at[j]], add=True)
      if compute_deg:
        pltpu.sync_copy(ones_v, deg_sh.at[dst_v.at[j]], add=True)

    plsc.subcore_barrier()

    @pl.loop(0, ROWS_PT // ZR)
    def _out(i):
      pltpu.sync_copy(acc_sh.at[pl.ds(row0 + i * ZR, ZR)], zbuf)
      pltpu.sync_copy(zbuf, out_hbm.at[cid, pl.ds(row0 + i * ZR, ZR)])

    if compute_deg:
      @pl.loop(0, ROWS_PT // ZR)
      def _outd(i):
        pltpu.sync_copy(deg_sh.at[pl.ds(row0 + i * ZR, ZR)], zdeg)
        pltpu.sync_copy(zdeg, deg_hbm.at[cid, pl.ds(row0 + i * ZR, ZR)])

  return pl.kernel(body, out_type=out_type, mesh=mesh, scratch_types=scratch,
                   name="sc_seg_sum_deg" if compute_deg else "sc_seg_sum")


_seg_sum_deg = _make_seg_sum(True)
_seg_sum = _make_seg_sum(False)


def _enc_body(x_ref, w_ref, b_ref, o_ref):
  o_ref[...] = jnp.maximum(
      jnp.dot(x_ref[...], w_ref[...], preferred_element_type=jnp.float32)
      + b_ref[...], 0.0)


def _encoder(x8, w8, b):
  return pl.pallas_call(
      _enc_body,
      grid=(GRID,),
      in_specs=[
          pl.BlockSpec((RB, 8), lambda i: (i, 0)),
          pl.BlockSpec((8, H), lambda i: (0, 0)),
          pl.BlockSpec((1, H), lambda i: (0, 0)),
      ],
      out_specs=pl.BlockSpec((RB, H), lambda i: (i, 0)),
      out_shape=jax.ShapeDtypeStruct((NP, H), jnp.float32),
  )(x8, w8, b)


def _upd_body(acc_ref, deg_ref, h_ref, wl_ref, bl_ref, wr_ref, g_ref, b_ref,
              o_ref):
  s = acc_ref[0] + acc_ref[1]
  deg = jnp.maximum(deg_ref[0, :, :1] + deg_ref[1, :, :1], 1.0)
  agg = s / deg
  h = h_ref[...]
  hn = (jnp.dot(agg, wl_ref[...], preferred_element_type=jnp.float32)
        + bl_ref[...]
        + jnp.dot(h, wr_ref[...], preferred_element_type=jnp.float32))
  mu = jnp.mean(hn, axis=-1, keepdims=True)
  var = jnp.mean((hn - mu) ** 2, axis=-1, keepdims=True)
  hn = (hn - mu) / jnp.sqrt(var + 1e-5) * g_ref[...] + b_ref[...]
  o_ref[...] = h + jnp.maximum(hn, 0.0)


def _update(acc, degp, h, wl, bl, wr, g, b):
  return pl.pallas_call(
      _upd_body,
      grid=(GRID,),
      in_specs=[
          pl.BlockSpec((NC, RB, H), lambda i: (0, i, 0)),
          pl.BlockSpec((NC, RB, 16), lambda i: (0, i, 0)),
          pl.BlockSpec((RB, H), lambda i: (i, 0)),
          pl.BlockSpec((H, H), lambda i: (0, 0)),
          pl.BlockSpec((1, H), lambda i: (0, 0)),
          pl.BlockSpec((H, H), lambda i: (0, 0)),
          pl.BlockSpec((1, H), lambda i: (0, 0)),
          pl.BlockSpec((1, H), lambda i: (0, 0)),
      ],
      out_specs=pl.BlockSpec((RB, H), lambda i: (i, 0)),
      out_shape=jax.ShapeDtypeStruct((NP, H), jnp.float32),
  )(acc, degp, h, wl, bl, wr, g, b)


def _pool_body(h_ref, bt_ref, tf_ref, tsW1_ref, tsb1_ref, tsg_ref, tsb_ref,
               tsW2_ref, tsb2_ref, g1_ref, g2_ref, g3_ref, b1_ref, b2_ref,
               b3_ref, W1a_ref, W1b_ref, W1c_ref, cb1_ref, cW2_ref, cb2_ref,
               o_ref, mean_acc, max_acc, cnt_acc):
  i = pl.program_id(0)

  @pl.when(i == 0)
  def _():
    mean_acc[...] = jnp.zeros_like(mean_acc)
    cnt_acc[...] = jnp.zeros_like(cnt_acc)
    max_acc[...] = jnp.full_like(max_acc, -jnp.inf)

  h = h_ref[...]
  bt = bt_ref[...]
  gids = lax.broadcasted_iota(jnp.int32, (RB, B), 1)
  mask = (bt == gids).astype(jnp.float32)
  mean_acc[...] += lax.dot_general(
      mask, h, (((0,), (0,)), ((), ())), preferred_element_type=jnp.float32)
  cnt = lax.dot_general(mask, jnp.ones((RB, 1), jnp.float32),
                        (((0,), (0,)), ((), ())),
                        preferred_element_type=jnp.float32)
  cnt_acc[...] += jnp.broadcast_to(cnt, (B, H))

  neg = jnp.float32(-jnp.inf)
  rows = [jnp.max(jnp.where(bt == g, h, neg), axis=0, keepdims=True)
          for g in range(B)]
  max_acc[...] = jnp.maximum(max_acc[...], jnp.concatenate(rows, axis=0))

  @pl.when(i == pl.num_programs(0) - 1)
  def _():
    cnt2 = jnp.maximum(cnt_acc[:, :1], 1.0)
    gm = mean_acc[...] / cnt2
    gx = max_acc[...]

    t = (jnp.dot(tf_ref[...], tsW1_ref[...], preferred_element_type=jnp.float32)
         + tsb1_ref[...])
    mu = jnp.mean(t, axis=-1, keepdims=True)
    var = jnp.mean((t - mu) ** 2, axis=-1, keepdims=True)
    t = (t - mu) / jnp.sqrt(var + 1e-5) * tsg_ref[...] + tsb_ref[...]
    t = jnp.maximum(t, 0.0)
    t = (jnp.dot(t, tsW2_ref[...], preferred_element_type=jnp.float32)
         + tsb2_ref[...])

    pool_w = jnp.float32(2 * H + H // 2)
    mu = (jnp.sum(gm, axis=-1, keepdims=True)
          + jnp.sum(gx, axis=-1, keepdims=True)
          + jnp.sum(t, axis=-1, keepdims=True)) / pool_w
    var = (jnp.sum((gm - mu) ** 2, axis=-1, keepdims=True)
           + jnp.sum((gx - mu) ** 2, axis=-1, keepdims=True)
           + jnp.sum((t - mu) ** 2, axis=-1, keepdims=True)) / pool_w
    sd = jnp.sqrt(var + 1e-5)
    z1 = (gm - mu) / sd * g1_ref[...] + b1_ref[...]
    z2 = (gx - mu) / sd * g2_ref[...] + b2_ref[...]
    z3 = (t - mu) / sd * g3_ref[...] + b3_ref[...]
    z = (jnp.dot(z1, W1a_ref[...], preferred_element_type=jnp.float32)
         + jnp.dot(z2, W1b_ref[...], preferred_element_type=jnp.float32)
         + jnp.dot(z3, W1c_ref[...], preferred_element_type=jnp.float32)
         + cb1_ref[...])
    z = jnp.maximum(z, 0.0)
    o_ref[...] = (jnp.dot(z, cW2_ref[...], preferred_element_type=jnp.float32)
                  + cb2_ref[...])


def _pool_classify(h, bt, tf8, tsW1, tsb1, tsg, tsb, tsW2, tsb2,
                   g1, g2, g3, b1, b2, b3, W1a, W1b, W1c, cb1, cW2, cb2):
  def full(shape):
    return pl.BlockSpec(shape, lambda *_: tuple(0 for _ in shape))
  return pl.pallas_call(
      _pool_body,
      grid=(GRID,),
      in_specs=[
          pl.BlockSpec((RB, H), lambda i: (i, 0)),
          pl.BlockSpec((RB, 1), lambda i: (i, 0)),
          full((B, 8)), full((8, H // 2)), full((1, H // 2)),
          full((1, H // 2)), full((1, H // 2)), full((H // 2, H // 2)),
          full((1, H // 2)),
          full((1, H)), full((1, H)), full((1, H // 2)),
          full((1, H)), full((1, H)), full((1, H // 2)),
          full((H, H)), full((H, H)), full((H // 2, H)),
          full((1, H)), full((H, NUM_CLASSES)), full((1, NUM_CLASSES)),
      ],
      out_specs=pl.BlockSpec((B, NUM_CLASSES), lambda i: (0, 0)),
      out_shape=jax.ShapeDtypeStruct((B, NUM_CLASSES), jnp.float32),
      scratch_shapes=[
          pltpu.VMEM((B, H), jnp.float32),
          pltpu.VMEM((B, H), jnp.float32),
          pltpu.VMEM((B, H), jnp.float32),
      ],
  )(h, bt, tf8, tsW1, tsb1, tsg, tsb, tsW2, tsb2,
    g1, g2, g3, b1, b2, b3, W1a, W1b, W1c, cb1, cW2, cb2)


def kernel(x, edge_index, batch, trackster_features, enc_W, enc_b, conv_Wl,
           conv_bl, conv_Wr, norm_g, norm_b, ts_W1, ts_b1, ts_ln_g, ts_ln_b,
           ts_W2, ts_b2, cls_ln_g, cls_ln_b, cls_W1, cls_b1, cls_W2, cls_b2):
  f32 = jnp.float32

  x8 = jnp.zeros((NP, 8), f32).at[:N, :F_IN].set(x.astype(f32))
  w8 = jnp.zeros((8, H), f32).at[:F_IN].set(enc_W.astype(f32))
  src2d = edge_index[0].astype(jnp.int32).reshape(NCH, CHUNK)
  dst2d = edge_index[1].astype(jnp.int32).reshape(NCH, CHUNK)
  bt = jnp.full((NP, 1), B, jnp.int32).at[:N, 0].set(batch.astype(jnp.int32))
  tf8 = jnp.zeros((B, 8), f32).at[:, :3].set(trackster_features.astype(f32))
  tsW1_8 = jnp.zeros((8, H // 2), f32).at[:3].set(ts_W1.astype(f32))

  g1 = cls_ln_g[None, :H]
  g2 = cls_ln_g[None, H:2 * H]
  g3 = cls_ln_g[None, 2 * H:]
  b1 = cls_ln_b[None, :H]
  b2 = cls_ln_b[None, H:2 * H]
  b3 = cls_ln_b[None, 2 * H:]
  W1a = cls_W1[:H]
  W1b = cls_W1[H:2 * H]
  W1c = cls_W1[2 * H:]

  h = _encoder(x8, w8, enc_b[None])

  degp = None
  for i in range(3):
    if i == 0:
      acc, degp = _seg_sum_deg(h, src2d, dst2d)
    else:
      acc = _seg_sum(h, src2d, dst2d)
    h = _update(acc, degp, h, conv_Wl[i], conv_bl[i][None], conv_Wr[i],
                norm_g[i][None], norm_b[i][None])

  return _pool_classify(
      h, bt, tf8, tsW1_8, ts_b1[None], ts_ln_g[None], ts_ln_b[None],
      ts_W2, ts_b2[None], g1, g2, g3, b1, b2, b3, W1a, W1b, W1c,
      cls_b1[None], cls_W2, cls_b2[None])


# speedup vs baseline: 4.5101x; 4.5101x over previous
import jax
import jax.numpy as jnp
from jax import lax
from jax.experimental import pallas as pl
from jax.experimental.pallas import tpu as pltpu
from jax.experimental.pallas import tpu_sc as plsc

N = 10000
E = 320000
B = 16
F_IN = 4
H = 128
NUM_CLASSES = 8

NP = 10240
NC = 2
NS = 16
NW = NC * NS
CHUNK = 40
NCH = E // CHUNK
NCH_W = NCH // NW
NB_I = 5
CH_B = NCH_W // NB_I
ROWS_PT = NP // NS
ZR = 32

RB = 512
GRID = NP // RB


def _make_seg_sum():
  mesh = plsc.VectorSubcoreMesh(core_axis_name="c", subcore_axis_name="s")
  out_type = jax.ShapeDtypeStruct((NC, NP, H), jnp.float32)

  scratch = [
      pltpu.VMEM((CH_B, CHUNK), jnp.int32),
      pltpu.VMEM((CH_B, CHUNK), jnp.int32),
      pltpu.VMEM((CHUNK, H), jnp.float32),
      pltpu.VMEM((ZR, H), jnp.float32),
      pltpu.VMEM_SHARED((NP, H), jnp.float32),
      pltpu.SemaphoreType.DMA,
  ]

  def body(h_hbm, src_hbm, dst_hbm, out_hbm, src_v, dst_v, rows_v, zbuf,
           acc_sh, sem):
    cid = lax.axis_index("c")
    sid = lax.axis_index("s")
    wid = sid * NC + cid
    row0 = sid * ROWS_PT

    @pl.loop(0, ZR)
    def _z(i):
      for c in range(H // 16):
        zbuf[i, pl.ds(c * 16, 16)] = jnp.zeros((16,), jnp.float32)

    @pl.loop(0, ROWS_PT // ZR)
    def _za(i):
      pltpu.sync_copy(zbuf, acc_sh.at[pl.ds(row0 + i * ZR, ZR)])

    plsc.subcore_barrier()

    @pl.loop(0, NB_I)
    def _blocks(ib):
      pltpu.sync_copy(src_hbm.at[wid, ib], src_v)
      pltpu.sync_copy(dst_hbm.at[wid, ib], dst_v)

      @pl.loop(0, CH_B)
      def _edges(j):
        pltpu.async_copy(h_hbm.at[src_v.at[j]], rows_v, sem).wait()
        pltpu.sync_copy(rows_v, acc_sh.at[dst_v.at[j]], add=True)

    plsc.subcore_barrier()

    @pl.loop(0, ROWS_PT // ZR)
    def _out(i):
      pltpu.sync_copy(acc_sh.at[pl.ds(row0 + i * ZR, ZR)], zbuf)
      pltpu.sync_copy(zbuf, out_hbm.at[cid, pl.ds(row0 + i * ZR, ZR)])

  return pl.kernel(body, out_type=out_type, mesh=mesh, scratch_types=scratch,
                   name="sc_seg_sum")


def _make_deg():
  mesh = plsc.VectorSubcoreMesh(core_axis_name="c", subcore_axis_name="s")
  out_type = jax.ShapeDtypeStruct((NC, NP, H), jnp.float32)

  scratch = [
      pltpu.VMEM((CH_B, CHUNK), jnp.int32),
      pltpu.VMEM((CHUNK, H), jnp.float32),
      pltpu.VMEM((ZR, H), jnp.float32),
      pltpu.VMEM_SHARED((NP, H), jnp.float32),
  ]

  def body(dst_hbm, out_hbm, dst_v, ones_v, zbuf, acc_sh):
    cid = lax.axis_index("c")
    sid = lax.axis_index("s")
    wid = sid * NC + cid
    row0 = sid * ROWS_PT

    @pl.loop(0, ZR)
    def _z(i):
      for c in range(H // 16):
        zbuf[i, pl.ds(c * 16, 16)] = jnp.zeros((16,), jnp.float32)

    @pl.loop(0, ROWS_PT // ZR)
    def _za(i):
      pltpu.sync_copy(zbuf, acc_sh.at[pl.ds(row0 + i * ZR, ZR)])

    @pl.loop(0, CHUNK)
    def _o(i):
      for c in range(H // 16):
        ones_v[i, pl.ds(c * 16, 16)] = jnp.ones((16,), jnp.float32)

    plsc.subcore_barrier()

    @pl.loop(0, NB_I)
    def _blocks(ib):
      pltpu.sync_copy(dst_hbm.at[wid, ib], dst_v)

      @pl.loop(0, CH_B)
      def _edges(j):
        pltpu.sync_copy(ones_v, acc_sh.at[dst_v.at[j]], add=True)

    plsc.subcore_barrier()

    @pl.loop(0, ROWS_PT // ZR)
    def _out(i):
      pltpu.sync_copy(acc_sh.at[pl.ds(row0 + i * ZR, ZR)], zbuf)
      pltpu.sync_copy(zbuf, out_hbm.at[cid, pl.ds(row0 + i * ZR, ZR)])

  return pl.kernel(body, out_type=out_type, mesh=mesh, scratch_types=scratch,
                   name="sc_deg")


_seg_sum = _make_seg_sum()
_deg_count = _make_deg()


def _enc_body(x_ref, w_ref, b_ref, o_ref):
  o_ref[...] = jnp.maximum(
      jnp.dot(x_ref[...], w_ref[...], preferred_element_type=jnp.float32)
      + b_ref[...], 0.0)


def _encoder(x8, w8, b):
  return pl.pallas_call(
      _enc_body,
      grid=(GRID,),
      in_specs=[
          pl.BlockSpec((RB, 8), lambda i: (i, 0)),
          pl.BlockSpec((8, H), lambda i: (0, 0)),
          pl.BlockSpec((1, H), lambda i: (0, 0)),
      ],
      out_specs=pl.BlockSpec((RB, H), lambda i: (i, 0)),
      out_shape=jax.ShapeDtypeStruct((NP, H), jnp.float32),
  )(x8, w8, b)


def _upd_body(acc_ref, deg_ref, h_ref, wl_ref, bl_ref, wr_ref, g_ref, b_ref,
              o_ref):
  s = acc_ref[0] + acc_ref[1]
  deg = jnp.maximum(deg_ref[0, :, :1] + deg_ref[1, :, :1], 1.0)
  agg = s / deg
  h = h_ref[...]
  hn = (jnp.dot(agg, wl_ref[...], preferred_element_type=jnp.float32)
        + bl_ref[...]
        + jnp.dot(h, wr_ref[...], preferred_element_type=jnp.float32))
  mu = jnp.mean(hn, axis=-1, keepdims=True)
  var = jnp.mean((hn - mu) ** 2, axis=-1, keepdims=True)
  hn = (hn - mu) / jnp.sqrt(var + 1e-5) * g_ref[...] + b_ref[...]
  o_ref[...] = h + jnp.maximum(hn, 0.0)


def _update(acc, degp, h, wl, bl, wr, g, b):
  return pl.pallas_call(
      _upd_body,
      grid=(GRID,),
      in_specs=[
          pl.BlockSpec((NC, RB, H), lambda i: (0, i, 0)),
          pl.BlockSpec((NC, RB, H), lambda i: (0, i, 0)),
          pl.BlockSpec((RB, H), lambda i: (i, 0)),
          pl.BlockSpec((H, H), lambda i: (0, 0)),
          pl.BlockSpec((1, H), lambda i: (0, 0)),
          pl.BlockSpec((H, H), lambda i: (0, 0)),
          pl.BlockSpec((1, H), lambda i: (0, 0)),
          pl.BlockSpec((1, H), lambda i: (0, 0)),
      ],
      out_specs=pl.BlockSpec((RB, H), lambda i: (i, 0)),
      out_shape=jax.ShapeDtypeStruct((NP, H), jnp.float32),
  )(acc, degp, h, wl, bl, wr, g, b)


def _pool_body(h_ref, bt_ref, tf_ref, tsW1_ref, tsb1_ref, tsg_ref, tsb_ref,
               tsW2_ref, tsb2_ref, g1_ref, g2_ref, g3_ref, b1_ref, b2_ref,
               b3_ref, W1a_ref, W1b_ref, W1c_ref, cb1_ref, cW2_ref, cb2_ref,
               o_ref, mean_acc, max_acc, cnt_acc):
  i = pl.program_id(0)

  @pl.when(i == 0)
  def _():
    mean_acc[...] = jnp.zeros_like(mean_acc)
    cnt_acc[...] = jnp.zeros_like(cnt_acc)
    max_acc[...] = jnp.full_like(max_acc, -jnp.inf)

  h = h_ref[...]
  bt = bt_ref[...]
  gids = lax.broadcasted_iota(jnp.int32, (RB, B), 1)
  mask = (bt == gids).astype(jnp.float32)
  mean_acc[...] += lax.dot_general(
      mask, h, (((0,), (0,)), ((), ())), preferred_element_type=jnp.float32)
  cnt = lax.dot_general(mask, jnp.ones((RB, 1), jnp.float32),
                        (((0,), (0,)), ((), ())),
                        preferred_element_type=jnp.float32)
  cnt_acc[...] += jnp.broadcast_to(cnt, (B, H))

  neg = jnp.float32(-jnp.inf)
  rows = [jnp.max(jnp.where(bt == g, h, neg), axis=0, keepdims=True)
          for g in range(B)]
  max_acc[...] = jnp.maximum(max_acc[...], jnp.concatenate(rows, axis=0))

  @pl.when(i == pl.num_programs(0) - 1)
  def _():
    cnt2 = jnp.maximum(cnt_acc[:, :1], 1.0)
    gm = mean_acc[...] / cnt2
    gx = max_acc[...]

    t = (jnp.dot(tf_ref[...], tsW1_ref[...], preferred_element_type=jnp.float32)
         + tsb1_ref[...])
    mu = jnp.mean(t, axis=-1, keepdims=True)
    var = jnp.mean((t - mu) ** 2, axis=-1, keepdims=True)
    t = (t - mu) / jnp.sqrt(var + 1e-5) * tsg_ref[...] + tsb_ref[...]
    t = jnp.maximum(t, 0.0)
    t = (jnp.dot(t, tsW2_ref[...], preferred_element_type=jnp.float32)
         + tsb2_ref[...])

    pool_w = jnp.float32(2 * H + H // 2)
    mu = (jnp.sum(gm, axis=-1, keepdims=True)
          + jnp.sum(gx, axis=-1, keepdims=True)
          + jnp.sum(t, axis=-1, keepdims=True)) / pool_w
    var = (jnp.sum((gm - mu) ** 2, axis=-1, keepdims=True)
           + jnp.sum((gx - mu) ** 2, axis=-1, keepdims=True)
           + jnp.sum((t - mu) ** 2, axis=-1, keepdims=True)) / pool_w
    sd = jnp.sqrt(var + 1e-5)
    z1 = (gm - mu) / sd * g1_ref[...] + b1_ref[...]
    z2 = (gx - mu) / sd * g2_ref[...] + b2_ref[...]
    z3 = (t - mu) / sd * g3_ref[...] + b3_ref[...]
    z = (jnp.dot(z1, W1a_ref[...], preferred_element_type=jnp.float32)
         + jnp.dot(z2, W1b_ref[...], preferred_element_type=jnp.float32)
         + jnp.dot(z3, W1c_ref[...], preferred_element_type=jnp.float32)
         + cb1_ref[...])
    z = jnp.maximum(z, 0.0)
    o_ref[...] = (jnp.dot(z, cW2_ref[...], preferred_element_type=jnp.float32)
                  + cb2_ref[...])


def _pool_classify(h, bt, tf8, tsW1, tsb1, tsg, tsb, tsW2, tsb2,
                   g1, g2, g3, b1, b2, b3, W1a, W1b, W1c, cb1, cW2, cb2):
  def full(shape):
    return pl.BlockSpec(shape, lambda *_: tuple(0 for _ in shape))
  return pl.pallas_call(
      _pool_body,
      grid=(GRID,),
      in_specs=[
          pl.BlockSpec((RB, H), lambda i: (i, 0)),
          pl.BlockSpec((RB, 1), lambda i: (i, 0)),
          full((B, 8)), full((8, H // 2)), full((1, H // 2)),
          full((1, H // 2)), full((1, H // 2)), full((H // 2, H // 2)),
          full((1, H // 2)),
          full((1, H)), full((1, H)), full((1, H // 2)),
          full((1, H)), full((1, H)), full((1, H // 2)),
          full((H, H)), full((H, H)), full((H // 2, H)),
          full((1, H)), full((H, NUM_CLASSES)), full((1, NUM_CLASSES)),
      ],
      out_specs=pl.BlockSpec((B, NUM_CLASSES), lambda i: (0, 0)),
      out_shape=jax.ShapeDtypeStruct((B, NUM_CLASSES), jnp.float32),
      scratch_shapes=[
          pltpu.VMEM((B, H), jnp.float32),
          pltpu.VMEM((B, H), jnp.float32),
          pltpu.VMEM((B, H), jnp.float32),
      ],
  )(h, bt, tf8, tsW1, tsb1, tsg, tsb, tsW2, tsb2,
    g1, g2, g3, b1, b2, b3, W1a, W1b, W1c, cb1, cW2, cb2)


def kernel(x, edge_index, batch, trackster_features, enc_W, enc_b, conv_Wl,
           conv_bl, conv_Wr, norm_g, norm_b, ts_W1, ts_b1, ts_ln_g, ts_ln_b,
           ts_W2, ts_b2, cls_ln_g, cls_ln_b, cls_W1, cls_b1, cls_W2, cls_b2):
  f32 = jnp.float32

  x8 = jnp.zeros((NP, 8), f32).at[:N, :F_IN].set(x.astype(f32))
  w8 = jnp.zeros((8, H), f32).at[:F_IN].set(enc_W.astype(f32))
  src2d = edge_index[0].astype(jnp.int32).reshape(NW, NB_I, CH_B, CHUNK)
  dst2d = edge_index[1].astype(jnp.int32).reshape(NW, NB_I, CH_B, CHUNK)
  bt = jnp.full((NP, 1), B, jnp.int32).at[:N, 0].set(batch.astype(jnp.int32))
  tf8 = jnp.zeros((B, 8), f32).at[:, :3].set(trackster_features.astype(f32))
  tsW1_8 = jnp.zeros((8, H // 2), f32).at[:3].set(ts_W1.astype(f32))

  g1 = cls_ln_g[None, :H]
  g2 = cls_ln_g[None, H:2 * H]
  g3 = cls_ln_g[None, 2 * H:]
  b1 = cls_ln_b[None, :H]
  b2 = cls_ln_b[None, H:2 * H]
  b3 = cls_ln_b[None, 2 * H:]
  W1a = cls_W1[:H]
  W1b = cls_W1[H:2 * H]
  W1c = cls_W1[2 * H:]

  h = _encoder(x8, w8, enc_b[None])

  degp = _deg_count(dst2d)
  for i in range(3):
    acc = _seg_sum(h, src2d, dst2d)
    h = _update(acc, degp, h, conv_Wl[i], conv_bl[i][None], conv_Wr[i],
                norm_g[i][None], norm_b[i][None])

  return _pool_classify(
      h, bt, tf8, tsW1_8, ts_b1[None], ts_ln_g[None], ts_ln_b[None],
      ts_W2, ts_b2[None], g1, g2, g3, b1, b2, b3, W1a, W1b, W1c,
      cls_b1[None], cls_W2, cls_b2[None])


# speedup vs baseline: 6.8543x; 1.5198x over previous
import jax
import jax.numpy as jnp
from jax import lax
from jax.experimental import pallas as pl
from jax.experimental.pallas import tpu as pltpu
from jax.experimental.pallas import tpu_sc as plsc

N = 10000
E = 320000
B = 16
F_IN = 4
H = 128
NUM_CLASSES = 8

NP = 10240
NC = 2
NS = 16
NW = NC * NS
CHUNK = 40
NCH = E // CHUNK
NCH_W = NCH // NW
NB_I = 5
CH_B = NCH_W // NB_I
ROWS_PT = NP // NS
ZR = 32

RB = 512
GRID = NP // RB


def _make_seg_sum():
  mesh = plsc.VectorSubcoreMesh(core_axis_name="c", subcore_axis_name="s")
  out_type = jax.ShapeDtypeStruct((NC, NP, H), jnp.float32)

  scratch = [
      pltpu.VMEM((CH_B, CHUNK), jnp.int32),
      pltpu.VMEM((CH_B, CHUNK), jnp.int32),
      pltpu.VMEM((CHUNK, H), jnp.float32),
      pltpu.VMEM((CHUNK, H), jnp.float32),
      pltpu.VMEM((ZR, H), jnp.float32),
      pltpu.VMEM_SHARED((NP, H), jnp.float32),
      pltpu.SemaphoreType.DMA,
      pltpu.SemaphoreType.DMA,
  ]

  def body(h_hbm, src_hbm, dst_hbm, out_hbm, src_v, dst_v, rows0, rows1,
           zbuf, acc_sh, sem0, sem1):
    cid = lax.axis_index("c")
    sid = lax.axis_index("s")
    wid = sid * NC + cid
    row0 = sid * ROWS_PT

    @pl.loop(0, ZR)
    def _z(i):
      for c in range(H // 16):
        zbuf[i, pl.ds(c * 16, 16)] = jnp.zeros((16,), jnp.float32)

    @pl.loop(0, ROWS_PT // ZR)
    def _za(i):
      pltpu.sync_copy(zbuf, acc_sh.at[pl.ds(row0 + i * ZR, ZR)])

    plsc.subcore_barrier()

    @pl.loop(0, NB_I)
    def _blocks(ib):
      pltpu.sync_copy(src_hbm.at[wid, ib], src_v)
      pltpu.sync_copy(dst_hbm.at[wid, ib], dst_v)

      pltpu.async_copy(h_hbm.at[src_v.at[0]], rows0, sem0)

      @pl.loop(0, CH_B // 2)
      def _pairs(k):
        j0 = 2 * k
        pltpu.async_copy(h_hbm.at[src_v.at[j0 + 1]], rows1, sem1)
        pltpu.make_async_copy(h_hbm.at[src_v.at[j0]], rows0, sem0).wait()
        pltpu.sync_copy(rows0, acc_sh.at[dst_v.at[j0]], add=True)

        @pl.when(k < CH_B // 2 - 1)
        def _():
          pltpu.async_copy(h_hbm.at[src_v.at[j0 + 2]], rows0, sem0)

        pltpu.make_async_copy(h_hbm.at[src_v.at[j0 + 1]], rows1, sem1).wait()
        pltpu.sync_copy(rows1, acc_sh.at[dst_v.at[j0 + 1]], add=True)

    plsc.subcore_barrier()

    @pl.loop(0, ROWS_PT // ZR)
    def _out(i):
      pltpu.sync_copy(acc_sh.at[pl.ds(row0 + i * ZR, ZR)], zbuf)
      pltpu.sync_copy(zbuf, out_hbm.at[cid, pl.ds(row0 + i * ZR, ZR)])

  return pl.kernel(body, out_type=out_type, mesh=mesh, scratch_types=scratch,
                   name="sc_seg_sum")


def _make_deg():
  mesh = plsc.VectorSubcoreMesh(core_axis_name="c", subcore_axis_name="s")
  out_type = jax.ShapeDtypeStruct((NC, NP, H), jnp.float32)

  scratch = [
      pltpu.VMEM((CH_B, CHUNK), jnp.int32),
      pltpu.VMEM((CHUNK, H), jnp.float32),
      pltpu.VMEM((ZR, H), jnp.float32),
      pltpu.VMEM_SHARED((NP, H), jnp.float32),
  ]

  def body(dst_hbm, out_hbm, dst_v, ones_v, zbuf, acc_sh):
    cid = lax.axis_index("c")
    sid = lax.axis_index("s")
    wid = sid * NC + cid
    row0 = sid * ROWS_PT

    @pl.loop(0, ZR)
    def _z(i):
      for c in range(H // 16):
        zbuf[i, pl.ds(c * 16, 16)] = jnp.zeros((16,), jnp.float32)

    @pl.loop(0, ROWS_PT // ZR)
    def _za(i):
      pltpu.sync_copy(zbuf, acc_sh.at[pl.ds(row0 + i * ZR, ZR)])

    @pl.loop(0, CHUNK)
    def _o(i):
      for c in range(H // 16):
        ones_v[i, pl.ds(c * 16, 16)] = jnp.ones((16,), jnp.float32)

    plsc.subcore_barrier()

    @pl.loop(0, NB_I)
    def _blocks(ib):
      pltpu.sync_copy(dst_hbm.at[wid, ib], dst_v)

      @pl.loop(0, CH_B)
      def _edges(j):
        pltpu.sync_copy(ones_v, acc_sh.at[dst_v.at[j]], add=True)

    plsc.subcore_barrier()

    @pl.loop(0, ROWS_PT // ZR)
    def _out(i):
      pltpu.sync_copy(acc_sh.at[pl.ds(row0 + i * ZR, ZR)], zbuf)
      pltpu.sync_copy(zbuf, out_hbm.at[cid, pl.ds(row0 + i * ZR, ZR)])

  return pl.kernel(body, out_type=out_type, mesh=mesh, scratch_types=scratch,
                   name="sc_deg")


_seg_sum = _make_seg_sum()
_deg_count = _make_deg()


def _enc_body(x_ref, w_ref, b_ref, o_ref):
  o_ref[...] = jnp.maximum(
      jnp.dot(x_ref[...], w_ref[...], preferred_element_type=jnp.float32)
      + b_ref[...], 0.0)


def _encoder(x8, w8, b):
  return pl.pallas_call(
      _enc_body,
      grid=(GRID,),
      in_specs=[
          pl.BlockSpec((RB, 8), lambda i: (i, 0)),
          pl.BlockSpec((8, H), lambda i: (0, 0)),
          pl.BlockSpec((1, H), lambda i: (0, 0)),
      ],
      out_specs=pl.BlockSpec((RB, H), lambda i: (i, 0)),
      out_shape=jax.ShapeDtypeStruct((NP, H), jnp.float32),
  )(x8, w8, b)


def _upd_body(acc_ref, deg_ref, h_ref, wl_ref, bl_ref, wr_ref, g_ref, b_ref,
              o_ref):
  s = acc_ref[0] + acc_ref[1]
  deg = jnp.maximum(deg_ref[0, :, :1] + deg_ref[1, :, :1], 1.0)
  agg = s / deg
  h = h_ref[...]
  hn = (jnp.dot(agg, wl_ref[...], preferred_element_type=jnp.float32)
        + bl_ref[...]
        + jnp.dot(h, wr_ref[...], preferred_element_type=jnp.float32))
  mu = jnp.mean(hn, axis=-1, keepdims=True)
  var = jnp.mean((hn - mu) ** 2, axis=-1, keepdims=True)
  hn = (hn - mu) / jnp.sqrt(var + 1e-5) * g_ref[...] + b_ref[...]
  o_ref[...] = h + jnp.maximum(hn, 0.0)


def _update(acc, degp, h, wl, bl, wr, g, b):
  return pl.pallas_call(
      _upd_body,
      grid=(GRID,),
      in_specs=[
          pl.BlockSpec((NC, RB, H), lambda i: (0, i, 0)),
          pl.BlockSpec((NC, RB, H), lambda i: (0, i, 0)),
          pl.BlockSpec((RB, H), lambda i: (i, 0)),
          pl.BlockSpec((H, H), lambda i: (0, 0)),
          pl.BlockSpec((1, H), lambda i: (0, 0)),
          pl.BlockSpec((H, H), lambda i: (0, 0)),
          pl.BlockSpec((1, H), lambda i: (0, 0)),
          pl.BlockSpec((1, H), lambda i: (0, 0)),
      ],
      out_specs=pl.BlockSpec((RB, H), lambda i: (i, 0)),
      out_shape=jax.ShapeDtypeStruct((NP, H), jnp.float32),
  )(acc, degp, h, wl, bl, wr, g, b)


def _pool_body(h_ref, bt_ref, tf_ref, tsW1_ref, tsb1_ref, tsg_ref, tsb_ref,
               tsW2_ref, tsb2_ref, g1_ref, g2_ref, g3_ref, b1_ref, b2_ref,
               b3_ref, W1a_ref, W1b_ref, W1c_ref, cb1_ref, cW2_ref, cb2_ref,
               o_ref, mean_acc, max_acc, cnt_acc):
  i = pl.program_id(0)

  @pl.when(i == 0)
  def _():
    mean_acc[...] = jnp.zeros_like(mean_acc)
    cnt_acc[...] = jnp.zeros_like(cnt_acc)
    max_acc[...] = jnp.full_like(max_acc, -jnp.inf)

  h = h_ref[...]
  bt = bt_ref[...]
  gids = lax.broadcasted_iota(jnp.int32, (RB, B), 1)
  mask = (bt == gids).astype(jnp.float32)
  mean_acc[...] += lax.dot_general(
      mask, h, (((0,), (0,)), ((), ())), preferred_element_type=jnp.float32)
  cnt = lax.dot_general(mask, jnp.ones((RB, 1), jnp.float32),
                        (((0,), (0,)), ((), ())),
                        preferred_element_type=jnp.float32)
  cnt_acc[...] += jnp.broadcast_to(cnt, (B, H))

  neg = jnp.float32(-jnp.inf)
  rows = [jnp.max(jnp.where(bt == g, h, neg), axis=0, keepdims=True)
          for g in range(B)]
  max_acc[...] = jnp.maximum(max_acc[...], jnp.concatenate(rows, axis=0))

  @pl.when(i == pl.num_programs(0) - 1)
  def _():
    cnt2 = jnp.maximum(cnt_acc[:, :1], 1.0)
    gm = mean_acc[...] / cnt2
    gx = max_acc[...]

    t = (jnp.dot(tf_ref[...], tsW1_ref[...], preferred_element_type=jnp.float32)
         + tsb1_ref[...])
    mu = jnp.mean(t, axis=-1, keepdims=True)
    var = jnp.mean((t - mu) ** 2, axis=-1, keepdims=True)
    t = (t - mu) / jnp.sqrt(var + 1e-5) * tsg_ref[...] + tsb_ref[...]
    t = jnp.maximum(t, 0.0)
    t = (jnp.dot(t, tsW2_ref[...], preferred_element_type=jnp.float32)
         + tsb2_ref[...])

    pool_w = jnp.float32(2 * H + H // 2)
    mu = (jnp.sum(gm, axis=-1, keepdims=True)
          + jnp.sum(gx, axis=-1, keepdims=True)
          + jnp.sum(t, axis=-1, keepdims=True)) / pool_w
    var = (jnp.sum((gm - mu) ** 2, axis=-1, keepdims=True)
           + jnp.sum((gx - mu) ** 2, axis=-1, keepdims=True)
           + jnp.sum((t - mu) ** 2, axis=-1, keepdims=True)) / pool_w
    sd = jnp.sqrt(var + 1e-5)
    z1 = (gm - mu) / sd * g1_ref[...] + b1_ref[...]
    z2 = (gx - mu) / sd * g2_ref[...] + b2_ref[...]
    z3 = (t - mu) / sd * g3_ref[...] + b3_ref[...]
    z = (jnp.dot(z1, W1a_ref[...], preferred_element_type=jnp.float32)
         + jnp.dot(z2, W1b_ref[...], preferred_element_type=jnp.float32)
         + jnp.dot(z3, W1c_ref[...], preferred_element_type=jnp.float32)
         + cb1_ref[...])
    z = jnp.maximum(z, 0.0)
    o_ref[...] = (jnp.dot(z, cW2_ref[...], preferred_element_type=jnp.float32)
                  + cb2_ref[...])


def _pool_classify(h, bt, tf8, tsW1, tsb1, tsg, tsb, tsW2, tsb2,
                   g1, g2, g3, b1, b2, b3, W1a, W1b, W1c, cb1, cW2, cb2):
  def full(shape):
    return pl.BlockSpec(shape, lambda *_: tuple(0 for _ in shape))
  return pl.pallas_call(
      _pool_body,
      grid=(GRID,),
      in_specs=[
          pl.BlockSpec((RB, H), lambda i: (i, 0)),
          pl.BlockSpec((RB, 1), lambda i: (i, 0)),
          full((B, 8)), full((8, H // 2)), full((1, H // 2)),
          full((1, H // 2)), full((1, H // 2)), full((H // 2, H // 2)),
          full((1, H // 2)),
          full((1, H)), full((1, H)), full((1, H // 2)),
          full((1, H)), full((1, H)), full((1, H // 2)),
          full((H, H)), full((H, H)), full((H // 2, H)),
          full((1, H)), full((H, NUM_CLASSES)), full((1, NUM_CLASSES)),
      ],
      out_specs=pl.BlockSpec((B, NUM_CLASSES), lambda i: (0, 0)),
      out_shape=jax.ShapeDtypeStruct((B, NUM_CLASSES), jnp.float32),
      scratch_shapes=[
          pltpu.VMEM((B, H), jnp.float32),
          pltpu.VMEM((B, H), jnp.float32),
          pltpu.VMEM((B, H), jnp.float32),
      ],
  )(h, bt, tf8, tsW1, tsb1, tsg, tsb, tsW2, tsb2,
    g1, g2, g3, b1, b2, b3, W1a, W1b, W1c, cb1, cW2, cb2)


def kernel(x, edge_index, batch, trackster_features, enc_W, enc_b, conv_Wl,
           conv_bl, conv_Wr, norm_g, norm_b, ts_W1, ts_b1, ts_ln_g, ts_ln_b,
           ts_W2, ts_b2, cls_ln_g, cls_ln_b, cls_W1, cls_b1, cls_W2, cls_b2):
  f32 = jnp.float32

  x8 = jnp.zeros((NP, 8), f32).at[:N, :F_IN].set(x.astype(f32))
  w8 = jnp.zeros((8, H), f32).at[:F_IN].set(enc_W.astype(f32))
  src2d = edge_index[0].astype(jnp.int32).reshape(NW, NB_I, CH_B, CHUNK)
  dst2d = edge_index[1].astype(jnp.int32).reshape(NW, NB_I, CH_B, CHUNK)
  bt = jnp.full((NP, 1), B, jnp.int32).at[:N, 0].set(batch.astype(jnp.int32))
  tf8 = jnp.zeros((B, 8), f32).at[:, :3].set(trackster_features.astype(f32))
  tsW1_8 = jnp.zeros((8, H // 2), f32).at[:3].set(ts_W1.astype(f32))

  g1 = cls_ln_g[None, :H]
  g2 = cls_ln_g[None, H:2 * H]
  g3 = cls_ln_g[None, 2 * H:]
  b1 = cls_ln_b[None, :H]
  b2 = cls_ln_b[None, H:2 * H]
  b3 = cls_ln_b[None, 2 * H:]
  W1a = cls_W1[:H]
  W1b = cls_W1[H:2 * H]
  W1c = cls_W1[2 * H:]

  h = _encoder(x8, w8, enc_b[None])

  degp = _deg_count(dst2d)
  for i in range(3):
    acc = _seg_sum(h, src2d, dst2d)
    h = _update(acc, degp, h, conv_Wl[i], conv_bl[i][None], conv_Wr[i],
                norm_g[i][None], norm_b[i][None])

  return _pool_classify(
      h, bt, tf8, tsW1_8, ts_b1[None], ts_ln_g[None], ts_ln_b[None],
      ts_W2, ts_b2[None], g1, g2, g3, b1, b2, b3, W1a, W1b, W1c,
      cls_b1[None], cls_W2, cls_b2[None])


# speedup vs baseline: 8.3125x; 1.2127x over previous
import jax
import jax.numpy as jnp
from jax import lax
from jax.experimental import pallas as pl
from jax.experimental.pallas import tpu as pltpu
from jax.experimental.pallas import tpu_sc as plsc

N = 10000
E = 320000
B = 16
F_IN = 4
H = 128
NUM_CLASSES = 8

NP = 10240
NC = 2
NS = 16
NW = NC * NS
CHUNK = 80
NCH = E // CHUNK
NCH_W = NCH // NW
NB_I = 5
CH_B = NCH_W // NB_I
ROWS_PT = NP // NS
ZR = 32

RB = 512
GRID = NP // RB


def _make_seg_sum():
  mesh = plsc.VectorSubcoreMesh(core_axis_name="c", subcore_axis_name="s")
  out_type = jax.ShapeDtypeStruct((NC, NP, H), jnp.float32)

  scratch = [
      pltpu.VMEM((CH_B, CHUNK), jnp.int32),
      pltpu.VMEM((CH_B, CHUNK), jnp.int32),
      pltpu.VMEM((CHUNK, H), jnp.float32),
      pltpu.VMEM((CHUNK, H), jnp.float32),
      pltpu.VMEM((ZR, H), jnp.float32),
      pltpu.VMEM_SHARED((NP, H), jnp.float32),
      pltpu.SemaphoreType.DMA,
      pltpu.SemaphoreType.DMA,
  ]

  def body(h_hbm, src_hbm, dst_hbm, out_hbm, src_v, dst_v, rows0, rows1,
           zbuf, acc_sh, sem0, sem1):
    cid = lax.axis_index("c")
    sid = lax.axis_index("s")
    wid = sid * NC + cid
    row0 = sid * ROWS_PT

    @pl.loop(0, ZR)
    def _z(i):
      for c in range(H // 16):
        zbuf[i, pl.ds(c * 16, 16)] = jnp.zeros((16,), jnp.float32)

    @pl.loop(0, ROWS_PT // ZR)
    def _za(i):
      pltpu.sync_copy(zbuf, acc_sh.at[pl.ds(row0 + i * ZR, ZR)])

    plsc.subcore_barrier()

    @pl.loop(0, NB_I)
    def _blocks(ib):
      pltpu.sync_copy(src_hbm.at[wid, ib], src_v)
      pltpu.sync_copy(dst_hbm.at[wid, ib], dst_v)

      pltpu.async_copy(h_hbm.at[src_v.at[0]], rows0, sem0)

      @pl.loop(0, CH_B // 2)
      def _pairs(k):
        j0 = 2 * k
        pltpu.async_copy(h_hbm.at[src_v.at[j0 + 1]], rows1, sem1)
        pltpu.make_async_copy(h_hbm.at[src_v.at[j0]], rows0, sem0).wait()
        pltpu.sync_copy(rows0, acc_sh.at[dst_v.at[j0]], add=True)
        pltpu.async_copy(h_hbm.at[src_v.at[j0 + 2]], rows0, sem0)
        pltpu.make_async_copy(h_hbm.at[src_v.at[j0 + 1]], rows1, sem1).wait()
        pltpu.sync_copy(rows1, acc_sh.at[dst_v.at[j0 + 1]], add=True)

      pltpu.make_async_copy(h_hbm.at[src_v.at[CH_B - 1]], rows0, sem0).wait()
      pltpu.sync_copy(rows0, acc_sh.at[dst_v.at[CH_B - 1]], add=True)

    plsc.subcore_barrier()

    @pl.loop(0, ROWS_PT // ZR)
    def _out(i):
      pltpu.sync_copy(acc_sh.at[pl.ds(row0 + i * ZR, ZR)], zbuf)
      pltpu.sync_copy(zbuf, out_hbm.at[cid, pl.ds(row0 + i * ZR, ZR)])

  return pl.kernel(body, out_type=out_type, mesh=mesh, scratch_types=scratch,
                   name="sc_seg_sum")


def _make_deg():
  mesh = plsc.VectorSubcoreMesh(core_axis_name="c", subcore_axis_name="s")
  out_type = jax.ShapeDtypeStruct((NC, NP, H), jnp.float32)

  scratch = [
      pltpu.VMEM((CH_B, CHUNK), jnp.int32),
      pltpu.VMEM((CHUNK, H), jnp.float32),
      pltpu.VMEM((ZR, H), jnp.float32),
      pltpu.VMEM_SHARED((NP, H), jnp.float32),
  ]

  def body(dst_hbm, out_hbm, dst_v, ones_v, zbuf, acc_sh):
    cid = lax.axis_index("c")
    sid = lax.axis_index("s")
    wid = sid * NC + cid
    row0 = sid * ROWS_PT

    @pl.loop(0, ZR)
    def _z(i):
      for c in range(H // 16):
        zbuf[i, pl.ds(c * 16, 16)] = jnp.zeros((16,), jnp.float32)

    @pl.loop(0, ROWS_PT // ZR)
    def _za(i):
      pltpu.sync_copy(zbuf, acc_sh.at[pl.ds(row0 + i * ZR, ZR)])

    @pl.loop(0, CHUNK)
    def _o(i):
      for c in range(H // 16):
        ones_v[i, pl.ds(c * 16, 16)] = jnp.ones((16,), jnp.float32)

    plsc.subcore_barrier()

    @pl.loop(0, NB_I)
    def _blocks(ib):
      pltpu.sync_copy(dst_hbm.at[wid, ib], dst_v)

      @pl.loop(0, CH_B)
      def _edges(j):
        pltpu.sync_copy(ones_v, acc_sh.at[dst_v.at[j]], add=True)

    plsc.subcore_barrier()

    @pl.loop(0, ROWS_PT // ZR)
    def _out(i):
      pltpu.sync_copy(acc_sh.at[pl.ds(row0 + i * ZR, ZR)], zbuf)
      pltpu.sync_copy(zbuf, out_hbm.at[cid, pl.ds(row0 + i * ZR, ZR)])

  return pl.kernel(body, out_type=out_type, mesh=mesh, scratch_types=scratch,
                   name="sc_deg")


_seg_sum = _make_seg_sum()
_deg_count = _make_deg()


def _enc_body(x_ref, w_ref, b_ref, o_ref):
  o_ref[...] = jnp.maximum(
      jnp.dot(x_ref[...], w_ref[...], preferred_element_type=jnp.float32)
      + b_ref[...], 0.0)


def _encoder(x8, w8, b):
  return pl.pallas_call(
      _enc_body,
      grid=(GRID,),
      in_specs=[
          pl.BlockSpec((RB, 8), lambda i: (i, 0)),
          pl.BlockSpec((8, H), lambda i: (0, 0)),
          pl.BlockSpec((1, H), lambda i: (0, 0)),
      ],
      out_specs=pl.BlockSpec((RB, H), lambda i: (i, 0)),
      out_shape=jax.ShapeDtypeStruct((NP, H), jnp.float32),
  )(x8, w8, b)


def _upd_body(acc_ref, deg_ref, h_ref, wl_ref, bl_ref, wr_ref, g_ref, b_ref,
              o_ref):
  s = acc_ref[0] + acc_ref[1]
  deg = jnp.maximum(deg_ref[0, :, :1] + deg_ref[1, :, :1], 1.0)
  agg = s / deg
  h = h_ref[...]
  hn = (jnp.dot(agg, wl_ref[...], preferred_element_type=jnp.float32)
        + bl_ref[...]
        + jnp.dot(h, wr_ref[...], preferred_element_type=jnp.float32))
  mu = jnp.mean(hn, axis=-1, keepdims=True)
  var = jnp.mean((hn - mu) ** 2, axis=-1, keepdims=True)
  hn = (hn - mu) / jnp.sqrt(var + 1e-5) * g_ref[...] + b_ref[...]
  o_ref[...] = h + jnp.maximum(hn, 0.0)


def _update(acc, degp, h, wl, bl, wr, g, b):
  return pl.pallas_call(
      _upd_body,
      grid=(GRID,),
      in_specs=[
          pl.BlockSpec((NC, RB, H), lambda i: (0, i, 0)),
          pl.BlockSpec((NC, RB, H), lambda i: (0, i, 0)),
          pl.BlockSpec((RB, H), lambda i: (i, 0)),
          pl.BlockSpec((H, H), lambda i: (0, 0)),
          pl.BlockSpec((1, H), lambda i: (0, 0)),
          pl.BlockSpec((H, H), lambda i: (0, 0)),
          pl.BlockSpec((1, H), lambda i: (0, 0)),
          pl.BlockSpec((1, H), lambda i: (0, 0)),
      ],
      out_specs=pl.BlockSpec((RB, H), lambda i: (i, 0)),
      out_shape=jax.ShapeDtypeStruct((NP, H), jnp.float32),
  )(acc, degp, h, wl, bl, wr, g, b)


def _pool_body(h_ref, bt_ref, tf_ref, tsW1_ref, tsb1_ref, tsg_ref, tsb_ref,
               tsW2_ref, tsb2_ref, g1_ref, g2_ref, g3_ref, b1_ref, b2_ref,
               b3_ref, W1a_ref, W1b_ref, W1c_ref, cb1_ref, cW2_ref, cb2_ref,
               o_ref, mean_acc, max_acc, cnt_acc):
  i = pl.program_id(0)

  @pl.when(i == 0)
  def _():
    mean_acc[...] = jnp.zeros_like(mean_acc)
    cnt_acc[...] = jnp.zeros_like(cnt_acc)
    max_acc[...] = jnp.full_like(max_acc, -jnp.inf)

  h = h_ref[...]
  bt = bt_ref[...]
  gids = lax.broadcasted_iota(jnp.int32, (RB, B), 1)
  mask = (bt == gids).astype(jnp.float32)
  mean_acc[...] += lax.dot_general(
      mask, h, (((0,), (0,)), ((), ())), preferred_element_type=jnp.float32)
  cnt = lax.dot_general(mask, jnp.ones((RB, 1), jnp.float32),
                        (((0,), (0,)), ((), ())),
                        preferred_element_type=jnp.float32)
  cnt_acc[...] += jnp.broadcast_to(cnt, (B, H))

  neg = jnp.float32(-jnp.inf)
  rows = [jnp.max(jnp.where(bt == g, h, neg), axis=0, keepdims=True)
          for g in range(B)]
  max_acc[...] = jnp.maximum(max_acc[...], jnp.concatenate(rows, axis=0))

  @pl.when(i == pl.num_programs(0) - 1)
  def _():
    cnt2 = jnp.maximum(cnt_acc[:, :1], 1.0)
    gm = mean_acc[...] / cnt2
    gx = max_acc[...]

    t = (jnp.dot(tf_ref[...], tsW1_ref[...], preferred_element_type=jnp.float32)
         + tsb1_ref[...])
    mu = jnp.mean(t, axis=-1, keepdims=True)
    var = jnp.mean((t - mu) ** 2, axis=-1, keepdims=True)
    t = (t - mu) / jnp.sqrt(var + 1e-5) * tsg_ref[...] + tsb_ref[...]
    t = jnp.maximum(t, 0.0)
    t = (jnp.dot(t, tsW2_ref[...], preferred_element_type=jnp.float32)
         + tsb2_ref[...])

    pool_w = jnp.float32(2 * H + H // 2)
    mu = (jnp.sum(gm, axis=-1, keepdims=True)
          + jnp.sum(gx, axis=-1, keepdims=True)
          + jnp.sum(t, axis=-1, keepdims=True)) / pool_w
    var = (jnp.sum((gm - mu) ** 2, axis=-1, keepdims=True)
           + jnp.sum((gx - mu) ** 2, axis=-1, keepdims=True)
           + jnp.sum((t - mu) ** 2, axis=-1, keepdims=True)) / pool_w
    sd = jnp.sqrt(var + 1e-5)
    z1 = (gm - mu) / sd * g1_ref[...] + b1_ref[...]
    z2 = (gx - mu) / sd * g2_ref[...] + b2_ref[...]
    z3 = (t - mu) / sd * g3_ref[...] + b3_ref[...]
    z = (jnp.dot(z1, W1a_ref[...], preferred_element_type=jnp.float32)
         + jnp.dot(z2, W1b_ref[...], preferred_element_type=jnp.float32)
         + jnp.dot(z3, W1c_ref[...], preferred_element_type=jnp.float32)
         + cb1_ref[...])
    z = jnp.maximum(z, 0.0)
    o_ref[...] = (jnp.dot(z, cW2_ref[...], preferred_element_type=jnp.float32)
                  + cb2_ref[...])


def _pool_classify(h, bt, tf8, tsW1, tsb1, tsg, tsb, tsW2, tsb2,
                   g1, g2, g3, b1, b2, b3, W1a, W1b, W1c, cb1, cW2, cb2):
  def full(shape):
    return pl.BlockSpec(shape, lambda *_: tuple(0 for _ in shape))
  return pl.pallas_call(
      _pool_body,
      grid=(GRID,),
      in_specs=[
          pl.BlockSpec((RB, H), lambda i: (i, 0)),
          pl.BlockSpec((RB, 1), lambda i: (i, 0)),
          full((B, 8)), full((8, H // 2)), full((1, H // 2)),
          full((1, H // 2)), full((1, H // 2)), full((H // 2, H // 2)),
          full((1, H // 2)),
          full((1, H)), full((1, H)), full((1, H // 2)),
          full((1, H)), full((1, H)), full((1, H // 2)),
          full((H, H)), full((H, H)), full((H // 2, H)),
          full((1, H)), full((H, NUM_CLASSES)), full((1, NUM_CLASSES)),
      ],
      out_specs=pl.BlockSpec((B, NUM_CLASSES), lambda i: (0, 0)),
      out_shape=jax.ShapeDtypeStruct((B, NUM_CLASSES), jnp.float32),
      scratch_shapes=[
          pltpu.VMEM((B, H), jnp.float32),
          pltpu.VMEM((B, H), jnp.float32),
          pltpu.VMEM((B, H), jnp.float32),
      ],
  )(h, bt, tf8, tsW1, tsb1, tsg, tsb, tsW2, tsb2,
    g1, g2, g3, b1, b2, b3, W1a, W1b, W1c, cb1, cW2, cb2)


def kernel(x, edge_index, batch, trackster_features, enc_W, enc_b, conv_Wl,
           conv_bl, conv_Wr, norm_g, norm_b, ts_W1, ts_b1, ts_ln_g, ts_ln_b,
           ts_W2, ts_b2, cls_ln_g, cls_ln_b, cls_W1, cls_b1, cls_W2, cls_b2):
  f32 = jnp.float32

  x8 = jnp.zeros((NP, 8), f32).at[:N, :F_IN].set(x.astype(f32))
  w8 = jnp.zeros((8, H), f32).at[:F_IN].set(enc_W.astype(f32))
  src2d = edge_index[0].astype(jnp.int32).reshape(NW, NB_I, CH_B, CHUNK)
  dst2d = edge_index[1].astype(jnp.int32).reshape(NW, NB_I, CH_B, CHUNK)
  bt = jnp.full((NP, 1), B, jnp.int32).at[:N, 0].set(batch.astype(jnp.int32))
  tf8 = jnp.zeros((B, 8), f32).at[:, :3].set(trackster_features.astype(f32))
  tsW1_8 = jnp.zeros((8, H // 2), f32).at[:3].set(ts_W1.astype(f32))

  g1 = cls_ln_g[None, :H]
  g2 = cls_ln_g[None, H:2 * H]
  g3 = cls_ln_g[None, 2 * H:]
  b1 = cls_ln_b[None, :H]
  b2 = cls_ln_b[None, H:2 * H]
  b3 = cls_ln_b[None, 2 * H:]
  W1a = cls_W1[:H]
  W1b = cls_W1[H:2 * H]
  W1c = cls_W1[2 * H:]

  h = _encoder(x8, w8, enc_b[None])

  degp = _deg_count(dst2d)
  for i in range(3):
    acc = _seg_sum(h, src2d, dst2d)
    h = _update(acc, degp, h, conv_Wl[i], conv_bl[i][None], conv_Wr[i],
                norm_g[i][None], norm_b[i][None])

  return _pool_classify(
      h, bt, tf8, tsW1_8, ts_b1[None], ts_ln_g[None], ts_ln_b[None],
      ts_W2, ts_b2[None], g1, g2, g3, b1, b2, b3, W1a, W1b, W1c,
      cls_b1[None], cls_W2, cls_b2[None])


# speedup vs baseline: 9.1397x; 1.0995x over previous
import jax
import jax.numpy as jnp
from jax import lax
from jax.experimental import pallas as pl
from jax.experimental.pallas import tpu as pltpu
from jax.experimental.pallas import tpu_sc as plsc

N = 10000
E = 320000
B = 16
F_IN = 4
H = 128
NUM_CLASSES = 8

NP = 10240
NC = 2
NS = 16
NW = NC * NS
CHUNK = 80
NCH = E // CHUNK
NCH_W = NCH // NW
NB_I = 5
CH_B = NCH_W // NB_I
ROWS_PT = NP // NS
ZR = 32

RB = 512
GRID = NP // RB


def _make_seg_sum():
  mesh = plsc.VectorSubcoreMesh(core_axis_name="c", subcore_axis_name="s")
  out_type = jax.ShapeDtypeStruct((NC, NP, H), jnp.float32)

  scratch = [
      pltpu.VMEM((CH_B, CHUNK), jnp.int32),
      pltpu.VMEM((CH_B, CHUNK), jnp.int32),
      pltpu.VMEM((CHUNK, H), jnp.float32),
      pltpu.VMEM((CHUNK, H), jnp.float32),
      pltpu.VMEM((CHUNK, H), jnp.float32),
      pltpu.VMEM((ZR, H), jnp.float32),
      pltpu.VMEM_SHARED((NP, H), jnp.float32),
      pltpu.SemaphoreType.DMA,
      pltpu.SemaphoreType.DMA,
      pltpu.SemaphoreType.DMA,
  ]

  def body(h_hbm, src_hbm, dst_hbm, out_hbm, src_v, dst_v, rows0, rows1,
           rows2, zbuf, acc_sh, sem0, sem1, sem2):
    cid = lax.axis_index("c")
    sid = lax.axis_index("s")
    wid = sid * NC + cid
    row0 = sid * ROWS_PT

    @pl.loop(0, ZR)
    def _z(i):
      for c in range(H // 16):
        zbuf[i, pl.ds(c * 16, 16)] = jnp.zeros((16,), jnp.float32)

    @pl.loop(0, ROWS_PT // ZR)
    def _za(i):
      pltpu.sync_copy(zbuf, acc_sh.at[pl.ds(row0 + i * ZR, ZR)])

    plsc.subcore_barrier()

    @pl.loop(0, NB_I)
    def _blocks(ib):
      pltpu.sync_copy(src_hbm.at[wid, ib], src_v)
      pltpu.sync_copy(dst_hbm.at[wid, ib], dst_v)

      bufs = ((rows0, sem0), (rows1, sem1), (rows2, sem2))
      for s in range(3):
        pltpu.async_copy(h_hbm.at[src_v.at[s]], bufs[s][0], bufs[s][1])

      @pl.loop(0, CH_B // 3)
      def _triples(k):
        for s in range(3):
          j = 3 * k + s
          rbuf, sem = bufs[s]
          pltpu.make_async_copy(h_hbm.at[src_v.at[j]], rbuf, sem).wait()
          pltpu.sync_copy(rbuf, acc_sh.at[dst_v.at[j]], add=True)

          @pl.when(j + 3 < CH_B)
          def _():
            pltpu.async_copy(h_hbm.at[src_v.at[j + 3]], rbuf, sem)

      pltpu.make_async_copy(h_hbm.at[src_v.at[CH_B - 1]], rows0, sem0).wait()
      pltpu.sync_copy(rows0, acc_sh.at[dst_v.at[CH_B - 1]], add=True)

    plsc.subcore_barrier()

    @pl.loop(0, ROWS_PT // ZR)
    def _out(i):
      pltpu.sync_copy(acc_sh.at[pl.ds(row0 + i * ZR, ZR)], zbuf)
      pltpu.sync_copy(zbuf, out_hbm.at[cid, pl.ds(row0 + i * ZR, ZR)])

  return pl.kernel(body, out_type=out_type, mesh=mesh, scratch_types=scratch,
                   name="sc_seg_sum")


def _make_deg():
  mesh = plsc.VectorSubcoreMesh(core_axis_name="c", subcore_axis_name="s")
  out_type = jax.ShapeDtypeStruct((NC, NP, H), jnp.float32)

  scratch = [
      pltpu.VMEM((CH_B, CHUNK), jnp.int32),
      pltpu.VMEM((CHUNK, H), jnp.float32),
      pltpu.VMEM((ZR, H), jnp.float32),
      pltpu.VMEM_SHARED((NP, H), jnp.float32),
  ]

  def body(dst_hbm, out_hbm, dst_v, ones_v, zbuf, acc_sh):
    cid = lax.axis_index("c")
    sid = lax.axis_index("s")
    wid = sid * NC + cid
    row0 = sid * ROWS_PT

    @pl.loop(0, ZR)
    def _z(i):
      for c in range(H // 16):
        zbuf[i, pl.ds(c * 16, 16)] = jnp.zeros((16,), jnp.float32)

    @pl.loop(0, ROWS_PT // ZR)
    def _za(i):
      pltpu.sync_copy(zbuf, acc_sh.at[pl.ds(row0 + i * ZR, ZR)])

    @pl.loop(0, CHUNK)
    def _o(i):
      for c in range(H // 16):
        ones_v[i, pl.ds(c * 16, 16)] = jnp.ones((16,), jnp.float32)

    plsc.subcore_barrier()

    @pl.loop(0, NB_I)
    def _blocks(ib):
      pltpu.sync_copy(dst_hbm.at[wid, ib], dst_v)

      @pl.loop(0, CH_B)
      def _edges(j):
        pltpu.sync_copy(ones_v, acc_sh.at[dst_v.at[j]], add=True)

    plsc.subcore_barrier()

    @pl.loop(0, ROWS_PT // ZR)
    def _out(i):
      pltpu.sync_copy(acc_sh.at[pl.ds(row0 + i * ZR, ZR)], zbuf)
      pltpu.sync_copy(zbuf, out_hbm.at[cid, pl.ds(row0 + i * ZR, ZR)])

  return pl.kernel(body, out_type=out_type, mesh=mesh, scratch_types=scratch,
                   name="sc_deg")


_seg_sum = _make_seg_sum()
_deg_count = _make_deg()


def _enc_body(x_ref, w_ref, b_ref, o_ref):
  o_ref[...] = jnp.maximum(
      jnp.dot(x_ref[...], w_ref[...], preferred_element_type=jnp.float32)
      + b_ref[...], 0.0)


def _encoder(x8, w8, b):
  return pl.pallas_call(
      _enc_body,
      grid=(GRID,),
      in_specs=[
          pl.BlockSpec((RB, 8), lambda i: (i, 0)),
          pl.BlockSpec((8, H), lambda i: (0, 0)),
          pl.BlockSpec((1, H), lambda i: (0, 0)),
      ],
      out_specs=pl.BlockSpec((RB, H), lambda i: (i, 0)),
      out_shape=jax.ShapeDtypeStruct((NP, H), jnp.float32),
  )(x8, w8, b)


def _upd_body(acc_ref, deg_ref, h_ref, wl_ref, bl_ref, wr_ref, g_ref, b_ref,
              o_ref):
  s = acc_ref[0] + acc_ref[1]
  deg = jnp.maximum(deg_ref[0, :, :1] + deg_ref[1, :, :1], 1.0)
  agg = s / deg
  h = h_ref[...]
  hn = (jnp.dot(agg, wl_ref[...], preferred_element_type=jnp.float32)
        + bl_ref[...]
        + jnp.dot(h, wr_ref[...], preferred_element_type=jnp.float32))
  mu = jnp.mean(hn, axis=-1, keepdims=True)
  var = jnp.mean((hn - mu) ** 2, axis=-1, keepdims=True)
  hn = (hn - mu) / jnp.sqrt(var + 1e-5) * g_ref[...] + b_ref[...]
  o_ref[...] = h + jnp.maximum(hn, 0.0)


def _update(acc, degp, h, wl, bl, wr, g, b):
  return pl.pallas_call(
      _upd_body,
      grid=(GRID,),
      in_specs=[
          pl.BlockSpec((NC, RB, H), lambda i: (0, i, 0)),
          pl.BlockSpec((NC, RB, H), lambda i: (0, i, 0)),
          pl.BlockSpec((RB, H), lambda i: (i, 0)),
          pl.BlockSpec((H, H), lambda i: (0, 0)),
          pl.BlockSpec((1, H), lambda i: (0, 0)),
          pl.BlockSpec((H, H), lambda i: (0, 0)),
          pl.BlockSpec((1, H), lambda i: (0, 0)),
          pl.BlockSpec((1, H), lambda i: (0, 0)),
      ],
      out_specs=pl.BlockSpec((RB, H), lambda i: (i, 0)),
      out_shape=jax.ShapeDtypeStruct((NP, H), jnp.float32),
  )(acc, degp, h, wl, bl, wr, g, b)


def _pool_body(h_ref, bt_ref, tf_ref, tsW1_ref, tsb1_ref, tsg_ref, tsb_ref,
               tsW2_ref, tsb2_ref, g1_ref, g2_ref, g3_ref, b1_ref, b2_ref,
               b3_ref, W1a_ref, W1b_ref, W1c_ref, cb1_ref, cW2_ref, cb2_ref,
               o_ref, mean_acc, max_acc, cnt_acc):
  i = pl.program_id(0)

  @pl.when(i == 0)
  def _():
    mean_acc[...] = jnp.zeros_like(mean_acc)
    cnt_acc[...] = jnp.zeros_like(cnt_acc)
    max_acc[...] = jnp.full_like(max_acc, -jnp.inf)

  h = h_ref[...]
  bt = bt_ref[...]
  gids = lax.broadcasted_iota(jnp.int32, (RB, B), 1)
  mask = (bt == gids).astype(jnp.float32)
  mean_acc[...] += lax.dot_general(
      mask, h, (((0,), (0,)), ((), ())), preferred_element_type=jnp.float32)
  cnt = lax.dot_general(mask, jnp.ones((RB, 1), jnp.float32),
                        (((0,), (0,)), ((), ())),
                        preferred_element_type=jnp.float32)
  cnt_acc[...] += jnp.broadcast_to(cnt, (B, H))

  neg = jnp.float32(-jnp.inf)
  rows = [jnp.max(jnp.where(bt == g, h, neg), axis=0, keepdims=True)
          for g in range(B)]
  max_acc[...] = jnp.maximum(max_acc[...], jnp.concatenate(rows, axis=0))

  @pl.when(i == pl.num_programs(0) - 1)
  def _():
    cnt2 = jnp.maximum(cnt_acc[:, :1], 1.0)
    gm = mean_acc[...] / cnt2
    gx = max_acc[...]

    t = (jnp.dot(tf_ref[...], tsW1_ref[...], preferred_element_type=jnp.float32)
         + tsb1_ref[...])
    mu = jnp.mean(t, axis=-1, keepdims=True)
    var = jnp.mean((t - mu) ** 2, axis=-1, keepdims=True)
    t = (t - mu) / jnp.sqrt(var + 1e-5) * tsg_ref[...] + tsb_ref[...]
    t = jnp.maximum(t, 0.0)
    t = (jnp.dot(t, tsW2_ref[...], preferred_element_type=jnp.float32)
         + tsb2_ref[...])

    pool_w = jnp.float32(2 * H + H // 2)
    mu = (jnp.sum(gm, axis=-1, keepdims=True)
          + jnp.sum(gx, axis=-1, keepdims=True)
          + jnp.sum(t, axis=-1, keepdims=True)) / pool_w
    var = (jnp.sum((gm - mu) ** 2, axis=-1, keepdims=True)
           + jnp.sum((gx - mu) ** 2, axis=-1, keepdims=True)
           + jnp.sum((t - mu) ** 2, axis=-1, keepdims=True)) / pool_w
    sd = jnp.sqrt(var + 1e-5)
    z1 = (gm - mu) / sd * g1_ref[...] + b1_ref[...]
    z2 = (gx - mu) / sd * g2_ref[...] + b2_ref[...]
    z3 = (t - mu) / sd * g3_ref[...] + b3_ref[...]
    z = (jnp.dot(z1, W1a_ref[...], preferred_element_type=jnp.float32)
         + jnp.dot(z2, W1b_ref[...], preferred_element_type=jnp.float32)
         + jnp.dot(z3, W1c_ref[...], preferred_element_type=jnp.float32)
         + cb1_ref[...])
    z = jnp.maximum(z, 0.0)
    o_ref[...] = (jnp.dot(z, cW2_ref[...], preferred_element_type=jnp.float32)
                  + cb2_ref[...])


def _pool_classify(h, bt, tf8, tsW1, tsb1, tsg, tsb, tsW2, tsb2,
                   g1, g2, g3, b1, b2, b3, W1a, W1b, W1c, cb1, cW2, cb2):
  def full(shape):
    return pl.BlockSpec(shape, lambda *_: tuple(0 for _ in shape))
  return pl.pallas_call(
      _pool_body,
      grid=(GRID,),
      in_specs=[
          pl.BlockSpec((RB, H), lambda i: (i, 0)),
          pl.BlockSpec((RB, 1), lambda i: (i, 0)),
          full((B, 8)), full((8, H // 2)), full((1, H // 2)),
          full((1, H // 2)), full((1, H // 2)), full((H // 2, H // 2)),
          full((1, H // 2)),
          full((1, H)), full((1, H)), full((1, H // 2)),
          full((1, H)), full((1, H)), full((1, H // 2)),
          full((H, H)), full((H, H)), full((H // 2, H)),
          full((1, H)), full((H, NUM_CLASSES)), full((1, NUM_CLASSES)),
      ],
      out_specs=pl.BlockSpec((B, NUM_CLASSES), lambda i: (0, 0)),
      out_shape=jax.ShapeDtypeStruct((B, NUM_CLASSES), jnp.float32),
      scratch_shapes=[
          pltpu.VMEM((B, H), jnp.float32),
          pltpu.VMEM((B, H), jnp.float32),
          pltpu.VMEM((B, H), jnp.float32),
      ],
  )(h, bt, tf8, tsW1, tsb1, tsg, tsb, tsW2, tsb2,
    g1, g2, g3, b1, b2, b3, W1a, W1b, W1c, cb1, cW2, cb2)


def kernel(x, edge_index, batch, trackster_features, enc_W, enc_b, conv_Wl,
           conv_bl, conv_Wr, norm_g, norm_b, ts_W1, ts_b1, ts_ln_g, ts_ln_b,
           ts_W2, ts_b2, cls_ln_g, cls_ln_b, cls_W1, cls_b1, cls_W2, cls_b2):
  f32 = jnp.float32

  x8 = jnp.zeros((NP, 8), f32).at[:N, :F_IN].set(x.astype(f32))
  w8 = jnp.zeros((8, H), f32).at[:F_IN].set(enc_W.astype(f32))
  src2d = edge_index[0].astype(jnp.int32).reshape(NW, NB_I, CH_B, CHUNK)
  dst2d = edge_index[1].astype(jnp.int32).reshape(NW, NB_I, CH_B, CHUNK)
  bt = jnp.full((NP, 1), B, jnp.int32).at[:N, 0].set(batch.astype(jnp.int32))
  tf8 = jnp.zeros((B, 8), f32).at[:, :3].set(trackster_features.astype(f32))
  tsW1_8 = jnp.zeros((8, H // 2), f32).at[:3].set(ts_W1.astype(f32))

  g1 = cls_ln_g[None, :H]
  g2 = cls_ln_g[None, H:2 * H]
  g3 = cls_ln_g[None, 2 * H:]
  b1 = cls_ln_b[None, :H]
  b2 = cls_ln_b[None, H:2 * H]
  b3 = cls_ln_b[None, 2 * H:]
  W1a = cls_W1[:H]
  W1b = cls_W1[H:2 * H]
  W1c = cls_W1[2 * H:]

  h = _encoder(x8, w8, enc_b[None])

  degp = _deg_count(dst2d)
  for i in range(3):
    acc = _seg_sum(h, src2d, dst2d)
    h = _update(acc, degp, h, conv_Wl[i], conv_bl[i][None], conv_Wr[i],
                norm_g[i][None], norm_b[i][None])

  return _pool_classify(
      h, bt, tf8, tsW1_8, ts_b1[None], ts_ln_g[None], ts_ln_b[None],
      ts_W2, ts_b2[None], g1, g2, g3, b1, b2, b3, W1a, W1b, W1c,
      cls_b1[None], cls_W2, cls_b2[None])


# speedup vs baseline: 9.1547x; 1.0016x over previous
import jax
import jax.numpy as jnp
from jax import lax
from jax.experimental import pallas as pl
from jax.experimental.pallas import tpu as pltpu
from jax.experimental.pallas import tpu_sc as plsc

N = 10000
E = 320000
B = 16
F_IN = 4
H = 128
NUM_CLASSES = 8

NP = 10240
NC = 2
NS = 16
NW = NC * NS
CHUNK = 80
NCH = E // CHUNK
NCH_W = NCH // NW
NB_I = 5
CH_B = NCH_W // NB_I
ROWS_PT = NP // NS
ZR = 32

RB = 512
GRID = NP // RB


def _make_seg_sum():
  mesh = plsc.VectorSubcoreMesh(core_axis_name="c", subcore_axis_name="s")
  out_type = jax.ShapeDtypeStruct((NC, NP, H), jnp.float32)

  scratch = [
      pltpu.VMEM((CH_B, CHUNK), jnp.int32),
      pltpu.VMEM((CH_B, CHUNK), jnp.int32),
      pltpu.VMEM((CHUNK, H), jnp.float32),
      pltpu.VMEM((CHUNK, H), jnp.float32),
      pltpu.VMEM((CHUNK, H), jnp.float32),
      pltpu.VMEM((ZR, H), jnp.float32),
      pltpu.VMEM_SHARED((NP, H), jnp.float32),
      pltpu.SemaphoreType.DMA,
      pltpu.SemaphoreType.DMA,
      pltpu.SemaphoreType.DMA,
  ]

  def body(h_hbm, src_hbm, dst_hbm, out_hbm, src_v, dst_v, rows0, rows1,
           rows2, zbuf, acc_sh, sem0, sem1, sem2):
    cid = lax.axis_index("c")
    sid = lax.axis_index("s")
    wid = sid * NC + cid
    row0 = sid * ROWS_PT

    @pl.loop(0, ZR)
    def _z(i):
      for c in range(H // 16):
        zbuf[i, pl.ds(c * 16, 16)] = jnp.zeros((16,), jnp.float32)

    @pl.loop(0, ROWS_PT // ZR)
    def _za(i):
      pltpu.sync_copy(zbuf, acc_sh.at[pl.ds(row0 + i * ZR, ZR)])

    plsc.subcore_barrier()

    @pl.loop(0, NB_I)
    def _blocks(ib):
      pltpu.sync_copy(src_hbm.at[wid, ib], src_v)
      pltpu.sync_copy(dst_hbm.at[wid, ib], dst_v)

      bufs = ((rows0, sem0), (rows1, sem1), (rows2, sem2))
      for s in range(3):
        pltpu.async_copy(h_hbm.at[src_v.at[s]], bufs[s][0], bufs[s][1])

      @pl.loop(0, CH_B // 3)
      def _triples(k):
        for s in range(3):
          j = 3 * k + s
          rbuf, sem = bufs[s]
          pltpu.make_async_copy(h_hbm.at[src_v.at[j]], rbuf, sem).wait()
          pltpu.sync_copy(rbuf, acc_sh.at[dst_v.at[j]], add=True)

          @pl.when(j + 3 < CH_B)
          def _():
            pltpu.async_copy(h_hbm.at[src_v.at[j + 3]], rbuf, sem)

      pltpu.make_async_copy(h_hbm.at[src_v.at[CH_B - 1]], rows0, sem0).wait()
      pltpu.sync_copy(rows0, acc_sh.at[dst_v.at[CH_B - 1]], add=True)

    plsc.subcore_barrier()

    @pl.loop(0, ROWS_PT // ZR)
    def _out(i):
      pltpu.sync_copy(acc_sh.at[pl.ds(row0 + i * ZR, ZR)], zbuf)
      pltpu.sync_copy(zbuf, out_hbm.at[cid, pl.ds(row0 + i * ZR, ZR)])

  return pl.kernel(body, out_type=out_type, mesh=mesh, scratch_types=scratch,
                   name="sc_seg_sum")


def _make_deg():
  mesh = plsc.VectorSubcoreMesh(core_axis_name="c", subcore_axis_name="s")
  out_type = jax.ShapeDtypeStruct((NC, NP, H), jnp.float32)

  scratch = [
      pltpu.VMEM((CH_B, CHUNK), jnp.int32),
      pltpu.VMEM((CHUNK, H), jnp.float32),
      pltpu.VMEM((ZR, H), jnp.float32),
      pltpu.VMEM_SHARED((NP, H), jnp.float32),
      pltpu.SemaphoreType.DMA,
  ]

  def body(dst_hbm, out_hbm, dst_v, ones_v, zbuf, acc_sh, sem):
    cid = lax.axis_index("c")
    sid = lax.axis_index("s")
    wid = sid * NC + cid
    row0 = sid * ROWS_PT

    @pl.loop(0, ZR)
    def _z(i):
      for c in range(H // 16):
        zbuf[i, pl.ds(c * 16, 16)] = jnp.zeros((16,), jnp.float32)

    @pl.loop(0, ROWS_PT // ZR)
    def _za(i):
      pltpu.sync_copy(zbuf, acc_sh.at[pl.ds(row0 + i * ZR, ZR)])

    @pl.loop(0, CHUNK)
    def _o(i):
      for c in range(H // 16):
        ones_v[i, pl.ds(c * 16, 16)] = jnp.ones((16,), jnp.float32)

    plsc.subcore_barrier()

    @pl.loop(0, NB_I)
    def _blocks(ib):
      pltpu.sync_copy(dst_hbm.at[wid, ib], dst_v)

      @pl.loop(0, CH_B)
      def _edges(j):
        pltpu.async_copy(ones_v, acc_sh.at[dst_v.at[j]], sem, add=True)

      @pl.loop(0, CH_B)
      def _drain(j):
        pltpu.make_async_copy(ones_v, acc_sh.at[dst_v.at[0]], sem).wait()

    plsc.subcore_barrier()

    @pl.loop(0, ROWS_PT // ZR)
    def _out(i):
      pltpu.sync_copy(acc_sh.at[pl.ds(row0 + i * ZR, ZR)], zbuf)
      pltpu.sync_copy(zbuf, out_hbm.at[cid, pl.ds(row0 + i * ZR, ZR)])

  return pl.kernel(body, out_type=out_type, mesh=mesh, scratch_types=scratch,
                   name="sc_deg")


_seg_sum = _make_seg_sum()
_deg_count = _make_deg()


def _enc_body(x_ref, w_ref, b_ref, o_ref):
  o_ref[...] = jnp.maximum(
      jnp.dot(x_ref[...], w_ref[...], preferred_element_type=jnp.float32)
      + b_ref[...], 0.0)


def _encoder(x8, w8, b):
  return pl.pallas_call(
      _enc_body,
      grid=(GRID,),
      in_specs=[
          pl.BlockSpec((RB, 8), lambda i: (i, 0)),
          pl.BlockSpec((8, H), lambda i: (0, 0)),
          pl.BlockSpec((1, H), lambda i: (0, 0)),
      ],
      out_specs=pl.BlockSpec((RB, H), lambda i: (i, 0)),
      out_shape=jax.ShapeDtypeStruct((NP, H), jnp.float32),
  )(x8, w8, b)


def _upd_body(acc_ref, deg_ref, h_ref, wl_ref, bl_ref, wr_ref, g_ref, b_ref,
              o_ref):
  s = acc_ref[0] + acc_ref[1]
  deg = jnp.maximum(deg_ref[0, :, :1] + deg_ref[1, :, :1], 1.0)
  agg = s / deg
  h = h_ref[...]
  hn = (jnp.dot(agg, wl_ref[...], preferred_element_type=jnp.float32)
        + bl_ref[...]
        + jnp.dot(h, wr_ref[...], preferred_element_type=jnp.float32))
  mu = jnp.mean(hn, axis=-1, keepdims=True)
  var = jnp.mean((hn - mu) ** 2, axis=-1, keepdims=True)
  hn = (hn - mu) / jnp.sqrt(var + 1e-5) * g_ref[...] + b_ref[...]
  o_ref[...] = h + jnp.maximum(hn, 0.0)


def _update(acc, degp, h, wl, bl, wr, g, b):
  return pl.pallas_call(
      _upd_body,
      grid=(GRID,),
      in_specs=[
          pl.BlockSpec((NC, RB, H), lambda i: (0, i, 0)),
          pl.BlockSpec((NC, RB, H), lambda i: (0, i, 0)),
          pl.BlockSpec((RB, H), lambda i: (i, 0)),
          pl.BlockSpec((H, H), lambda i: (0, 0)),
          pl.BlockSpec((1, H), lambda i: (0, 0)),
          pl.BlockSpec((H, H), lambda i: (0, 0)),
          pl.BlockSpec((1, H), lambda i: (0, 0)),
          pl.BlockSpec((1, H), lambda i: (0, 0)),
      ],
      out_specs=pl.BlockSpec((RB, H), lambda i: (i, 0)),
      out_shape=jax.ShapeDtypeStruct((NP, H), jnp.float32),
  )(acc, degp, h, wl, bl, wr, g, b)


def _pool_body(h_ref, bt_ref, tf_ref, tsW1_ref, tsb1_ref, tsg_ref, tsb_ref,
               tsW2_ref, tsb2_ref, g1_ref, g2_ref, g3_ref, b1_ref, b2_ref,
               b3_ref, W1a_ref, W1b_ref, W1c_ref, cb1_ref, cW2_ref, cb2_ref,
               o_ref, mean_acc, max_acc, cnt_acc):
  i = pl.program_id(0)

  @pl.when(i == 0)
  def _():
    mean_acc[...] = jnp.zeros_like(mean_acc)
    cnt_acc[...] = jnp.zeros_like(cnt_acc)
    max_acc[...] = jnp.full_like(max_acc, -jnp.inf)

  h = h_ref[...]
  bt = bt_ref[...]
  gids = lax.broadcasted_iota(jnp.int32, (RB, B), 1)
  mask = (bt == gids).astype(jnp.float32)
  mean_acc[...] += lax.dot_general(
      mask, h, (((0,), (0,)), ((), ())), preferred_element_type=jnp.float32)
  cnt = lax.dot_general(mask, jnp.ones((RB, 1), jnp.float32),
                        (((0,), (0,)), ((), ())),
                        preferred_element_type=jnp.float32)
  cnt_acc[...] += jnp.broadcast_to(cnt, (B, H))

  neg = jnp.float32(-jnp.inf)
  rows = [jnp.max(jnp.where(bt == g, h, neg), axis=0, keepdims=True)
          for g in range(B)]
  max_acc[...] = jnp.maximum(max_acc[...], jnp.concatenate(rows, axis=0))

  @pl.when(i == pl.num_programs(0) - 1)
  def _():
    cnt2 = jnp.maximum(cnt_acc[:, :1], 1.0)
    gm = mean_acc[...] / cnt2
    gx = max_acc[...]

    t = (jnp.dot(tf_ref[...], tsW1_ref[...], preferred_element_type=jnp.float32)
         + tsb1_ref[...])
    mu = jnp.mean(t, axis=-1, keepdims=True)
    var = jnp.mean((t - mu) ** 2, axis=-1, keepdims=True)
    t = (t - mu) / jnp.sqrt(var + 1e-5) * tsg_ref[...] + tsb_ref[...]
    t = jnp.maximum(t, 0.0)
    t = (jnp.dot(t, tsW2_ref[...], preferred_element_type=jnp.float32)
         + tsb2_ref[...])

    pool_w = jnp.float32(2 * H + H // 2)
    mu = (jnp.sum(gm, axis=-1, keepdims=True)
          + jnp.sum(gx, axis=-1, keepdims=True)
          + jnp.sum(t, axis=-1, keepdims=True)) / pool_w
    var = (jnp.sum((gm - mu) ** 2, axis=-1, keepdims=True)
           + jnp.sum((gx - mu) ** 2, axis=-1, keepdims=True)
           + jnp.sum((t - mu) ** 2, axis=-1, keepdims=True)) / pool_w
    sd = jnp.sqrt(var + 1e-5)
    z1 = (gm - mu) / sd * g1_ref[...] + b1_ref[...]
    z2 = (gx - mu) / sd * g2_ref[...] + b2_ref[...]
    z3 = (t - mu) / sd * g3_ref[...] + b3_ref[...]
    z = (jnp.dot(z1, W1a_ref[...], preferred_element_type=jnp.float32)
         + jnp.dot(z2, W1b_ref[...], preferred_element_type=jnp.float32)
         + jnp.dot(z3, W1c_ref[...], preferred_element_type=jnp.float32)
         + cb1_ref[...])
    z = jnp.maximum(z, 0.0)
    o_ref[...] = (jnp.dot(z, cW2_ref[...], preferred_element_type=jnp.float32)
                  + cb2_ref[...])


def _pool_classify(h, bt, tf8, tsW1, tsb1, tsg, tsb, tsW2, tsb2,
                   g1, g2, g3, b1, b2, b3, W1a, W1b, W1c, cb1, cW2, cb2):
  def full(shape):
    return pl.BlockSpec(shape, lambda *_: tuple(0 for _ in shape))
  return pl.pallas_call(
      _pool_body,
      grid=(GRID,),
      in_specs=[
          pl.BlockSpec((RB, H), lambda i: (i, 0)),
          pl.BlockSpec((RB, 1), lambda i: (i, 0)),
          full((B, 8)), full((8, H // 2)), full((1, H // 2)),
          full((1, H // 2)), full((1, H // 2)), full((H // 2, H // 2)),
          full((1, H // 2)),
          full((1, H)), full((1, H)), full((1, H // 2)),
          full((1, H)), full((1, H)), full((1, H // 2)),
          full((H, H)), full((H, H)), full((H // 2, H)),
          full((1, H)), full((H, NUM_CLASSES)), full((1, NUM_CLASSES)),
      ],
      out_specs=pl.BlockSpec((B, NUM_CLASSES), lambda i: (0, 0)),
      out_shape=jax.ShapeDtypeStruct((B, NUM_CLASSES), jnp.float32),
      scratch_shapes=[
          pltpu.VMEM((B, H), jnp.float32),
          pltpu.VMEM((B, H), jnp.float32),
          pltpu.VMEM((B, H), jnp.float32),
      ],
  )(h, bt, tf8, tsW1, tsb1, tsg, tsb, tsW2, tsb2,
    g1, g2, g3, b1, b2, b3, W1a, W1b, W1c, cb1, cW2, cb2)


def kernel(x, edge_index, batch, trackster_features, enc_W, enc_b, conv_Wl,
           conv_bl, conv_Wr, norm_g, norm_b, ts_W1, ts_b1, ts_ln_g, ts_ln_b,
           ts_W2, ts_b2, cls_ln_g, cls_ln_b, cls_W1, cls_b1, cls_W2, cls_b2):
  f32 = jnp.float32

  x8 = jnp.zeros((NP, 8), f32).at[:N, :F_IN].set(x.astype(f32))
  w8 = jnp.zeros((8, H), f32).at[:F_IN].set(enc_W.astype(f32))
  src2d = edge_index[0].astype(jnp.int32).reshape(NW, NB_I, CH_B, CHUNK)
  dst2d = edge_index[1].astype(jnp.int32).reshape(NW, NB_I, CH_B, CHUNK)
  bt = jnp.full((NP, 1), B, jnp.int32).at[:N, 0].set(batch.astype(jnp.int32))
  tf8 = jnp.zeros((B, 8), f32).at[:, :3].set(trackster_features.astype(f32))
  tsW1_8 = jnp.zeros((8, H // 2), f32).at[:3].set(ts_W1.astype(f32))

  g1 = cls_ln_g[None, :H]
  g2 = cls_ln_g[None, H:2 * H]
  g3 = cls_ln_g[None, 2 * H:]
  b1 = cls_ln_b[None, :H]
  b2 = cls_ln_b[None, H:2 * H]
  b3 = cls_ln_b[None, 2 * H:]
  W1a = cls_W1[:H]
  W1b = cls_W1[H:2 * H]
  W1c = cls_W1[2 * H:]

  h = _encoder(x8, w8, enc_b[None])

  degp = _deg_count(dst2d)
  for i in range(3):
    acc = _seg_sum(h, src2d, dst2d)
    h = _update(acc, degp, h, conv_Wl[i], conv_bl[i][None], conv_Wr[i],
                norm_g[i][None], norm_b[i][None])

  return _pool_classify(
      h, bt, tf8, tsW1_8, ts_b1[None], ts_ln_g[None], ts_ln_b[None],
      ts_W2, ts_b2[None], g1, g2, g3, b1, b2, b3, W1a, W1b, W1c,
      cls_b1[None], cls_W2, cls_b2[None])


# speedup vs baseline: 9.2390x; 1.0092x over previous
import jax
import jax.numpy as jnp
from jax import lax
from jax.experimental import pallas as pl
from jax.experimental.pallas import tpu as pltpu
from jax.experimental.pallas import tpu_sc as plsc

N = 10000
E = 320000
B = 16
F_IN = 4
H = 128
NUM_CLASSES = 8

NP = 10240
NC = 2
NS = 16
NW = NC * NS
CHUNK = 80
NCH = E // CHUNK
NCH_W = NCH // NW
NB_I = 5
CH_B = NCH_W // NB_I
ROWS_PT = NP // NS
ZR = 32

RB = 512
GRID = NP // RB


def _make_seg_sum():
  mesh = plsc.VectorSubcoreMesh(core_axis_name="c", subcore_axis_name="s")
  out_type = jax.ShapeDtypeStruct((NC, NP, H), jnp.float32)

  scratch = [
      pltpu.VMEM((CH_B, CHUNK), jnp.int32),
      pltpu.VMEM((CH_B, CHUNK), jnp.int32),
      pltpu.VMEM((CHUNK, H), jnp.float32),
      pltpu.VMEM((CHUNK, H), jnp.float32),
      pltpu.VMEM((CHUNK, H), jnp.float32),
      pltpu.VMEM((ZR, H), jnp.float32),
      pltpu.VMEM_SHARED((NP, H), jnp.float32),
      pltpu.SemaphoreType.DMA,
      pltpu.SemaphoreType.DMA,
      pltpu.SemaphoreType.DMA,
  ]

  def body(h_hbm, src_hbm, dst_hbm, out_hbm, src_v, dst_v, rows0, rows1,
           rows2, zbuf, acc_sh, sem0, sem1, sem2):
    cid = lax.axis_index("c")
    sid = lax.axis_index("s")
    wid = sid * NC + cid
    row0 = sid * ROWS_PT
    bufs = ((rows0, sem0), (rows1, sem1), (rows2, sem2))

    pltpu.sync_copy(src_hbm.at[wid, 0], src_v)
    pltpu.sync_copy(dst_hbm.at[wid, 0], dst_v)
    for s in range(3):
      pltpu.async_copy(h_hbm.at[src_v.at[s]], bufs[s][0], bufs[s][1])

    @pl.loop(0, ZR)
    def _z(i):
      for c in range(H // 16):
        zbuf[i, pl.ds(c * 16, 16)] = jnp.zeros((16,), jnp.float32)

    @pl.loop(0, ROWS_PT // ZR)
    def _za(i):
      pltpu.sync_copy(zbuf, acc_sh.at[pl.ds(row0 + i * ZR, ZR)])

    plsc.subcore_barrier()

    @pl.loop(0, NB_I)
    def _blocks(ib):
      @pl.when(ib > 0)
      def _():
        pltpu.sync_copy(src_hbm.at[wid, ib], src_v)
        pltpu.sync_copy(dst_hbm.at[wid, ib], dst_v)
        for s in range(3):
          pltpu.async_copy(h_hbm.at[src_v.at[s]], bufs[s][0], bufs[s][1])

      @pl.loop(0, CH_B // 3)
      def _triples(k):
        for s in range(3):
          j = 3 * k + s
          rbuf, sem = bufs[s]
          pltpu.make_async_copy(h_hbm.at[src_v.at[j]], rbuf, sem).wait()
          pltpu.sync_copy(rbuf, acc_sh.at[dst_v.at[j]], add=True)

          @pl.when(j + 3 < CH_B)
          def _():
            pltpu.async_copy(h_hbm.at[src_v.at[j + 3]], rbuf, sem)

      pltpu.make_async_copy(h_hbm.at[src_v.at[CH_B - 1]], rows0, sem0).wait()
      pltpu.sync_copy(rows0, acc_sh.at[dst_v.at[CH_B - 1]], add=True)

    plsc.subcore_barrier()

    @pl.loop(0, ROWS_PT // ZR)
    def _out(i):
      pltpu.sync_copy(acc_sh.at[pl.ds(row0 + i * ZR, ZR)], zbuf)
      pltpu.sync_copy(zbuf, out_hbm.at[cid, pl.ds(row0 + i * ZR, ZR)])

  return pl.kernel(body, out_type=out_type, mesh=mesh, scratch_types=scratch,
                   name="sc_seg_sum")


def _make_deg():
  mesh = plsc.VectorSubcoreMesh(core_axis_name="c", subcore_axis_name="s")
  out_type = jax.ShapeDtypeStruct((NC, NP, H), jnp.float32)

  scratch = [
      pltpu.VMEM((CH_B, CHUNK), jnp.int32),
      pltpu.VMEM((CHUNK, H), jnp.float32),
      pltpu.VMEM((ZR, H), jnp.float32),
      pltpu.VMEM_SHARED((NP, H), jnp.float32),
      pltpu.SemaphoreType.DMA,
  ]

  def body(dst_hbm, out_hbm, dst_v, ones_v, zbuf, acc_sh, sem):
    cid = lax.axis_index("c")
    sid = lax.axis_index("s")
    wid = sid * NC + cid
    row0 = sid * ROWS_PT

    @pl.loop(0, ZR)
    def _z(i):
      for c in range(H // 16):
        zbuf[i, pl.ds(c * 16, 16)] = jnp.zeros((16,), jnp.float32)

    @pl.loop(0, ROWS_PT // ZR)
    def _za(i):
      pltpu.sync_copy(zbuf, acc_sh.at[pl.ds(row0 + i * ZR, ZR)])

    @pl.loop(0, CHUNK)
    def _o(i):
      for c in range(H // 16):
        ones_v[i, pl.ds(c * 16, 16)] = jnp.ones((16,), jnp.float32)

    plsc.subcore_barrier()

    @pl.loop(0, NB_I)
    def _blocks(ib):
      pltpu.sync_copy(dst_hbm.at[wid, ib], dst_v)

      @pl.loop(0, CH_B)
      def _edges(j):
        pltpu.async_copy(ones_v, acc_sh.at[dst_v.at[j]], sem, add=True)

      @pl.loop(0, CH_B)
      def _drain(j):
        pltpu.make_async_copy(ones_v, acc_sh.at[dst_v.at[0]], sem).wait()

    plsc.subcore_barrier()

    @pl.loop(0, ROWS_PT // ZR)
    def _out(i):
      pltpu.sync_copy(acc_sh.at[pl.ds(row0 + i * ZR, ZR)], zbuf)
      pltpu.sync_copy(zbuf, out_hbm.at[cid, pl.ds(row0 + i * ZR, ZR)])

  return pl.kernel(body, out_type=out_type, mesh=mesh, scratch_types=scratch,
                   name="sc_deg")


_seg_sum = _make_seg_sum()
_deg_count = _make_deg()


def _enc_body(x_ref, w_ref, b_ref, o_ref):
  o_ref[...] = jnp.maximum(
      jnp.dot(x_ref[...], w_ref[...], preferred_element_type=jnp.float32)
      + b_ref[...], 0.0)


def _encoder(x8, w8, b):
  return pl.pallas_call(
      _enc_body,
      grid=(GRID,),
      in_specs=[
          pl.BlockSpec((RB, 8), lambda i: (i, 0)),
          pl.BlockSpec((8, H), lambda i: (0, 0)),
          pl.BlockSpec((1, H), lambda i: (0, 0)),
      ],
      out_specs=pl.BlockSpec((RB, H), lambda i: (i, 0)),
      out_shape=jax.ShapeDtypeStruct((NP, H), jnp.float32),
  )(x8, w8, b)


def _upd_body(acc_ref, deg_ref, h_ref, wl_ref, bl_ref, wr_ref, g_ref, b_ref,
              o_ref):
  s = acc_ref[0] + acc_ref[1]
  deg = jnp.maximum(deg_ref[0, :, :1] + deg_ref[1, :, :1], 1.0)
  agg = s / deg
  h = h_ref[...]
  hn = (jnp.dot(agg, wl_ref[...], preferred_element_type=jnp.float32)
        + bl_ref[...]
        + jnp.dot(h, wr_ref[...], preferred_element_type=jnp.float32))
  mu = jnp.mean(hn, axis=-1, keepdims=True)
  var = jnp.mean((hn - mu) ** 2, axis=-1, keepdims=True)
  hn = (hn - mu) / jnp.sqrt(var + 1e-5) * g_ref[...] + b_ref[...]
  o_ref[...] = h + jnp.maximum(hn, 0.0)


def _update(acc, degp, h, wl, bl, wr, g, b):
  return pl.pallas_call(
      _upd_body,
      grid=(GRID,),
      in_specs=[
          pl.BlockSpec((NC, RB, H), lambda i: (0, i, 0)),
          pl.BlockSpec((NC, RB, H), lambda i: (0, i, 0)),
          pl.BlockSpec((RB, H), lambda i: (i, 0)),
          pl.BlockSpec((H, H), lambda i: (0, 0)),
          pl.BlockSpec((1, H), lambda i: (0, 0)),
          pl.BlockSpec((H, H), lambda i: (0, 0)),
          pl.BlockSpec((1, H), lambda i: (0, 0)),
          pl.BlockSpec((1, H), lambda i: (0, 0)),
      ],
      out_specs=pl.BlockSpec((RB, H), lambda i: (i, 0)),
      out_shape=jax.ShapeDtypeStruct((NP, H), jnp.float32),
  )(acc, degp, h, wl, bl, wr, g, b)


def _pool_body(h_ref, bt_ref, tf_ref, tsW1_ref, tsb1_ref, tsg_ref, tsb_ref,
               tsW2_ref, tsb2_ref, g1_ref, g2_ref, g3_ref, b1_ref, b2_ref,
               b3_ref, W1a_ref, W1b_ref, W1c_ref, cb1_ref, cW2_ref, cb2_ref,
               o_ref, mean_acc, max_acc, cnt_acc):
  i = pl.program_id(0)

  @pl.when(i == 0)
  def _():
    mean_acc[...] = jnp.zeros_like(mean_acc)
    cnt_acc[...] = jnp.zeros_like(cnt_acc)
    max_acc[...] = jnp.full_like(max_acc, -jnp.inf)

  h = h_ref[...]
  bt = bt_ref[...]
  gids = lax.broadcasted_iota(jnp.int32, (RB, B), 1)
  mask = (bt == gids).astype(jnp.float32)
  mean_acc[...] += lax.dot_general(
      mask, h, (((0,), (0,)), ((), ())), preferred_element_type=jnp.float32)
  cnt = lax.dot_general(mask, jnp.ones((RB, 1), jnp.float32),
                        (((0,), (0,)), ((), ())),
                        preferred_element_type=jnp.float32)
  cnt_acc[...] += jnp.broadcast_to(cnt, (B, H))

  neg = jnp.float32(-jnp.inf)
  rows = [jnp.max(jnp.where(bt == g, h, neg), axis=0, keepdims=True)
          for g in range(B)]
  max_acc[...] = jnp.maximum(max_acc[...], jnp.concatenate(rows, axis=0))

  @pl.when(i == pl.num_programs(0) - 1)
  def _():
    cnt2 = jnp.maximum(cnt_acc[:, :1], 1.0)
    gm = mean_acc[...] / cnt2
    gx = max_acc[...]

    t = (jnp.dot(tf_ref[...], tsW1_ref[...], preferred_element_type=jnp.float32)
         + tsb1_ref[...])
    mu = jnp.mean(t, axis=-1, keepdims=True)
    var = jnp.mean((t - mu) ** 2, axis=-1, keepdims=True)
    t = (t - mu) / jnp.sqrt(var + 1e-5) * tsg_ref[...] + tsb_ref[...]
    t = jnp.maximum(t, 0.0)
    t = (jnp.dot(t, tsW2_ref[...], preferred_element_type=jnp.float32)
         + tsb2_ref[...])

    pool_w = jnp.float32(2 * H + H // 2)
    mu = (jnp.sum(gm, axis=-1, keepdims=True)
          + jnp.sum(gx, axis=-1, keepdims=True)
          + jnp.sum(t, axis=-1, keepdims=True)) / pool_w
    var = (jnp.sum((gm - mu) ** 2, axis=-1, keepdims=True)
           + jnp.sum((gx - mu) ** 2, axis=-1, keepdims=True)
           + jnp.sum((t - mu) ** 2, axis=-1, keepdims=True)) / pool_w
    sd = jnp.sqrt(var + 1e-5)
    z1 = (gm - mu) / sd * g1_ref[...] + b1_ref[...]
    z2 = (gx - mu) / sd * g2_ref[...] + b2_ref[...]
    z3 = (t - mu) / sd * g3_ref[...] + b3_ref[...]
    z = (jnp.dot(z1, W1a_ref[...], preferred_element_type=jnp.float32)
         + jnp.dot(z2, W1b_ref[...], preferred_element_type=jnp.float32)
         + jnp.dot(z3, W1c_ref[...], preferred_element_type=jnp.float32)
         + cb1_ref[...])
    z = jnp.maximum(z, 0.0)
    o_ref[...] = (jnp.dot(z, cW2_ref[...], preferred_element_type=jnp.float32)
                  + cb2_ref[...])


def _pool_classify(h, bt, tf8, tsW1, tsb1, tsg, tsb, tsW2, tsb2,
                   g1, g2, g3, b1, b2, b3, W1a, W1b, W1c, cb1, cW2, cb2):
  def full(shape):
    return pl.BlockSpec(shape, lambda *_: tuple(0 for _ in shape))
  return pl.pallas_call(
      _pool_body,
      grid=(GRID,),
      in_specs=[
          pl.BlockSpec((RB, H), lambda i: (i, 0)),
          pl.BlockSpec((RB, 1), lambda i: (i, 0)),
          full((B, 8)), full((8, H // 2)), full((1, H // 2)),
          full((1, H // 2)), full((1, H // 2)), full((H // 2, H // 2)),
          full((1, H // 2)),
          full((1, H)), full((1, H)), full((1, H // 2)),
          full((1, H)), full((1, H)), full((1, H // 2)),
          full((H, H)), full((H, H)), full((H // 2, H)),
          full((1, H)), full((H, NUM_CLASSES)), full((1, NUM_CLASSES)),
      ],
      out_specs=pl.BlockSpec((B, NUM_CLASSES), lambda i: (0, 0)),
      out_shape=jax.ShapeDtypeStruct((B, NUM_CLASSES), jnp.float32),
      scratch_shapes=[
          pltpu.VMEM((B, H), jnp.float32),
          pltpu.VMEM((B, H), jnp.float32),
          pltpu.VMEM((B, H), jnp.float32),
      ],
  )(h, bt, tf8, tsW1, tsb1, tsg, tsb, tsW2, tsb2,
    g1, g2, g3, b1, b2, b3, W1a, W1b, W1c, cb1, cW2, cb2)


def kernel(x, edge_index, batch, trackster_features, enc_W, enc_b, conv_Wl,
           conv_bl, conv_Wr, norm_g, norm_b, ts_W1, ts_b1, ts_ln_g, ts_ln_b,
           ts_W2, ts_b2, cls_ln_g, cls_ln_b, cls_W1, cls_b1, cls_W2, cls_b2):
  f32 = jnp.float32

  x8 = jnp.zeros((NP, 8), f32).at[:N, :F_IN].set(x.astype(f32))
  w8 = jnp.zeros((8, H), f32).at[:F_IN].set(enc_W.astype(f32))
  src2d = edge_index[0].astype(jnp.int32).reshape(NW, NB_I, CH_B, CHUNK)
  dst2d = edge_index[1].astype(jnp.int32).reshape(NW, NB_I, CH_B, CHUNK)
  bt = jnp.full((NP, 1), B, jnp.int32).at[:N, 0].set(batch.astype(jnp.int32))
  tf8 = jnp.zeros((B, 8), f32).at[:, :3].set(trackster_features.astype(f32))
  tsW1_8 = jnp.zeros((8, H // 2), f32).at[:3].set(ts_W1.astype(f32))

  g1 = cls_ln_g[None, :H]
  g2 = cls_ln_g[None, H:2 * H]
  g3 = cls_ln_g[None, 2 * H:]
  b1 = cls_ln_b[None, :H]
  b2 = cls_ln_b[None, H:2 * H]
  b3 = cls_ln_b[None, 2 * H:]
  W1a = cls_W1[:H]
  W1b = cls_W1[H:2 * H]
  W1c = cls_W1[2 * H:]

  h = _encoder(x8, w8, enc_b[None])

  degp = _deg_count(dst2d)
  for i in range(3):
    acc = _seg_sum(h, src2d, dst2d)
    h = _update(acc, degp, h, conv_Wl[i], conv_bl[i][None], conv_Wr[i],
                norm_g[i][None], norm_b[i][None])

  return _pool_classify(
      h, bt, tf8, tsW1_8, ts_b1[None], ts_ln_g[None], ts_ln_b[None],
      ts_W2, ts_b2[None], g1, g2, g3, b1, b2, b3, W1a, W1b, W1c,
      cls_b1[None], cls_W2, cls_b2[None])


# speedup vs baseline: 9.4693x; 1.0249x over previous
import jax
import jax.numpy as jnp
from jax import lax
from jax.experimental import pallas as pl
from jax.experimental.pallas import tpu as pltpu
from jax.experimental.pallas import tpu_sc as plsc

N = 10000
E = 320000
B = 16
F_IN = 4
H = 128
NUM_CLASSES = 8

NP = 10240
NC = 2
NS = 16
NW = NC * NS
CHUNK = 80
NCH = E // CHUNK
NCH_W = NCH // NW
NB_I = 5
CH_B = NCH_W // NB_I
ROWS_PT = NP // NS
ZR = 32

RB = 512
GRID = NP // RB


def _make_seg_sum():
  mesh = plsc.VectorSubcoreMesh(core_axis_name="c", subcore_axis_name="s")
  out_type = jax.ShapeDtypeStruct((NC, NP, H), jnp.float32)

  scratch = [
      pltpu.VMEM((CH_B, CHUNK), jnp.int32),
      pltpu.VMEM((CH_B, CHUNK), jnp.int32),
      pltpu.VMEM((CHUNK, H), jnp.float32),
      pltpu.VMEM((CHUNK, H), jnp.float32),
      pltpu.VMEM((CHUNK, H), jnp.float32),
      pltpu.VMEM((ZR, H), jnp.float32),
      pltpu.VMEM_SHARED((NP, H), jnp.float32),
      pltpu.SemaphoreType.DMA,
      pltpu.SemaphoreType.DMA,
      pltpu.SemaphoreType.DMA,
  ]

  def body(h_hbm, src_hbm, dst_hbm, out_hbm, src_v, dst_v, rows0, rows1,
           rows2, zbuf, acc_sh, sem0, sem1, sem2):
    cid = lax.axis_index("c")
    sid = lax.axis_index("s")
    wid = sid * NC + cid
    row0 = sid * ROWS_PT
    bufs = ((rows0, sem0), (rows1, sem1), (rows2, sem2))

    pltpu.sync_copy(src_hbm.at[wid, 0], src_v)
    pltpu.sync_copy(dst_hbm.at[wid, 0], dst_v)
    for s in range(3):
      pltpu.async_copy(h_hbm.at[src_v.at[s]], bufs[s][0], bufs[s][1])

    @pl.loop(0, ZR)
    def _z(i):
      for c in range(H // 16):
        zbuf[i, pl.ds(c * 16, 16)] = jnp.zeros((16,), jnp.float32)

    @pl.loop(0, ROWS_PT // ZR)
    def _za(i):
      pltpu.sync_copy(zbuf, acc_sh.at[pl.ds(row0 + i * ZR, ZR)])

    plsc.subcore_barrier()

    @pl.loop(0, NB_I)
    def _blocks(ib):
      @pl.when(ib > 0)
      def _():
        pltpu.sync_copy(src_hbm.at[wid, ib], src_v)
        pltpu.sync_copy(dst_hbm.at[wid, ib], dst_v)
        for s in range(3):
          pltpu.async_copy(h_hbm.at[src_v.at[s]], bufs[s][0], bufs[s][1])

      @pl.loop(0, CH_B // 3)
      def _triples(k):
        for s in range(3):
          j = 3 * k + s
          rbuf, sem = bufs[s]
          pltpu.make_async_copy(h_hbm.at[src_v.at[j]], rbuf, sem).wait()
          pltpu.sync_copy(rbuf, acc_sh.at[dst_v.at[j]], add=True)

          @pl.when(j + 3 < CH_B)
          def _():
            pltpu.async_copy(h_hbm.at[src_v.at[j + 3]], rbuf, sem)

      pltpu.make_async_copy(h_hbm.at[src_v.at[CH_B - 1]], rows0, sem0).wait()
      pltpu.sync_copy(rows0, acc_sh.at[dst_v.at[CH_B - 1]], add=True)

    plsc.subcore_barrier()

    @pl.loop(0, ROWS_PT // ZR)
    def _out(i):
      pltpu.sync_copy(acc_sh.at[pl.ds(row0 + i * ZR, ZR)], zbuf)
      pltpu.sync_copy(zbuf, out_hbm.at[cid, pl.ds(row0 + i * ZR, ZR)])

  return pl.kernel(body, out_type=out_type, mesh=mesh, scratch_types=scratch,
                   name="sc_seg_sum")


def _make_deg():
  mesh = plsc.VectorSubcoreMesh(core_axis_name="c", subcore_axis_name="s")
  out_type = jax.ShapeDtypeStruct((NC, NP, H), jnp.float32)

  scratch = [
      pltpu.VMEM((CH_B, CHUNK), jnp.int32),
      pltpu.VMEM((CHUNK, H), jnp.float32),
      pltpu.VMEM((ZR, H), jnp.float32),
      pltpu.VMEM_SHARED((NP, H), jnp.float32),
      pltpu.SemaphoreType.DMA,
  ]

  def body(dst_hbm, out_hbm, dst_v, ones_v, zbuf, acc_sh, sem):
    cid = lax.axis_index("c")
    sid = lax.axis_index("s")
    wid = sid * NC + cid
    row0 = sid * ROWS_PT

    @pl.loop(0, ZR)
    def _z(i):
      for c in range(H // 16):
        zbuf[i, pl.ds(c * 16, 16)] = jnp.zeros((16,), jnp.float32)

    @pl.loop(0, ROWS_PT // ZR)
    def _za(i):
      pltpu.sync_copy(zbuf, acc_sh.at[pl.ds(row0 + i * ZR, ZR)])

    @pl.loop(0, CHUNK)
    def _o(i):
      for c in range(H // 16):
        ones_v[i, pl.ds(c * 16, 16)] = jnp.ones((16,), jnp.float32)

    plsc.subcore_barrier()

    @pl.loop(0, NB_I)
    def _blocks(ib):
      pltpu.sync_copy(dst_hbm.at[wid, ib], dst_v)

      @pl.loop(0, CH_B)
      def _edges(j):
        pltpu.async_copy(ones_v, acc_sh.at[dst_v.at[j]], sem, add=True)

      @pl.loop(0, CH_B)
      def _drain(j):
        pltpu.make_async_copy(ones_v, acc_sh.at[dst_v.at[0]], sem).wait()

    plsc.subcore_barrier()

    @pl.loop(0, ROWS_PT // ZR)
    def _out(i):
      pltpu.sync_copy(acc_sh.at[pl.ds(row0 + i * ZR, ZR)], zbuf)
      pltpu.sync_copy(zbuf, out_hbm.at[cid, pl.ds(row0 + i * ZR, ZR)])

  return pl.kernel(body, out_type=out_type, mesh=mesh, scratch_types=scratch,
                   name="sc_deg")


_seg_sum = _make_seg_sum()
_deg_count = _make_deg()


def _enc_body(x_ref, w_ref, b_ref, o_ref):
  o_ref[...] = jnp.maximum(
      jnp.dot(x_ref[...], w_ref[...], preferred_element_type=jnp.float32)
      + b_ref[...], 0.0)


def _encoder(x8, w8, b):
  return pl.pallas_call(
      _enc_body,
      grid=(GRID,),
      in_specs=[
          pl.BlockSpec((RB, 8), lambda i: (i, 0)),
          pl.BlockSpec((8, H), lambda i: (0, 0)),
          pl.BlockSpec((1, H), lambda i: (0, 0)),
      ],
      out_specs=pl.BlockSpec((RB, H), lambda i: (i, 0)),
      out_shape=jax.ShapeDtypeStruct((NP, H), jnp.float32),
  )(x8, w8, b)


def _upd_body(acc_ref, deg_ref, h_ref, wl_ref, bl_ref, wr_ref, g_ref, b_ref,
              o_ref):
  s = acc_ref[0] + acc_ref[1]
  deg = jnp.maximum(deg_ref[0, :, :1] + deg_ref[1, :, :1], 1.0)
  agg = s / deg
  h = h_ref[...]
  hn = (jnp.dot(agg, wl_ref[...], preferred_element_type=jnp.float32)
        + bl_ref[...]
        + jnp.dot(h, wr_ref[...], preferred_element_type=jnp.float32))
  mu = jnp.mean(hn, axis=-1, keepdims=True)
  var = jnp.mean((hn - mu) ** 2, axis=-1, keepdims=True)
  hn = (hn - mu) / jnp.sqrt(var + 1e-5) * g_ref[...] + b_ref[...]
  o_ref[...] = h + jnp.maximum(hn, 0.0)


def _update(acc, degp, h, wl, bl, wr, g, b):
  return pl.pallas_call(
      _upd_body,
      grid=(GRID,),
      in_specs=[
          pl.BlockSpec((NC, RB, H), lambda i: (0, i, 0)),
          pl.BlockSpec((NC, RB, H), lambda i: (0, i, 0)),
          pl.BlockSpec((RB, H), lambda i: (i, 0)),
          pl.BlockSpec((H, H), lambda i: (0, 0)),
          pl.BlockSpec((1, H), lambda i: (0, 0)),
          pl.BlockSpec((H, H), lambda i: (0, 0)),
          pl.BlockSpec((1, H), lambda i: (0, 0)),
          pl.BlockSpec((1, H), lambda i: (0, 0)),
      ],
      out_specs=pl.BlockSpec((RB, H), lambda i: (i, 0)),
      out_shape=jax.ShapeDtypeStruct((NP, H), jnp.float32),
  )(acc, degp, h, wl, bl, wr, g, b)


def _pool_body(acc_ref, deg_ref, h_ref, wl_ref, bl_ref, wr_ref, g_ref, b_ref,
               bt_ref, tf_ref, tsW1_ref, tsb1_ref, tsg_ref, tsb_ref,
               tsW2_ref, tsb2_ref, g1_ref, g2_ref, g3_ref, b1_ref, b2_ref,
               b3_ref, W1a_ref, W1b_ref, W1c_ref, cb1_ref, cW2_ref, cb2_ref,
               o_ref, mean_acc, max_acc, cnt_acc):
  i = pl.program_id(0)

  @pl.when(i == 0)
  def _():
    mean_acc[...] = jnp.zeros_like(mean_acc)
    cnt_acc[...] = jnp.zeros_like(cnt_acc)
    max_acc[...] = jnp.full_like(max_acc, -jnp.inf)

  s = acc_ref[0] + acc_ref[1]
  deg = jnp.maximum(deg_ref[0, :, :1] + deg_ref[1, :, :1], 1.0)
  agg = s / deg
  hp = h_ref[...]
  hn = (jnp.dot(agg, wl_ref[...], preferred_element_type=jnp.float32)
        + bl_ref[...]
        + jnp.dot(hp, wr_ref[...], preferred_element_type=jnp.float32))
  mu = jnp.mean(hn, axis=-1, keepdims=True)
  var = jnp.mean((hn - mu) ** 2, axis=-1, keepdims=True)
  hn = (hn - mu) / jnp.sqrt(var + 1e-5) * g_ref[...] + b_ref[...]
  h = hp + jnp.maximum(hn, 0.0)

  bt = bt_ref[...]
  gids = lax.broadcasted_iota(jnp.int32, (RB, B), 1)
  mask = (bt == gids).astype(jnp.float32)
  mean_acc[...] += lax.dot_general(
      mask, h, (((0,), (0,)), ((), ())), preferred_element_type=jnp.float32)
  cnt = lax.dot_general(mask, jnp.ones((RB, 1), jnp.float32),
                        (((0,), (0,)), ((), ())),
                        preferred_element_type=jnp.float32)
  cnt_acc[...] += jnp.broadcast_to(cnt, (B, H))

  neg = jnp.float32(-jnp.inf)
  rows = [jnp.max(jnp.where(bt == g, h, neg), axis=0, keepdims=True)
          for g in range(B)]
  max_acc[...] = jnp.maximum(max_acc[...], jnp.concatenate(rows, axis=0))

  @pl.when(i == pl.num_programs(0) - 1)
  def _():
    cnt2 = jnp.maximum(cnt_acc[:, :1], 1.0)
    gm = mean_acc[...] / cnt2
    gx = max_acc[...]

    t = (jnp.dot(tf_ref[...], tsW1_ref[...], preferred_element_type=jnp.float32)
         + tsb1_ref[...])
    mu = jnp.mean(t, axis=-1, keepdims=True)
    var = jnp.mean((t - mu) ** 2, axis=-1, keepdims=True)
    t = (t - mu) / jnp.sqrt(var + 1e-5) * tsg_ref[...] + tsb_ref[...]
    t = jnp.maximum(t, 0.0)
    t = (jnp.dot(t, tsW2_ref[...], preferred_element_type=jnp.float32)
         + tsb2_ref[...])

    pool_w = jnp.float32(2 * H + H // 2)
    mu = (jnp.sum(gm, axis=-1, keepdims=True)
          + jnp.sum(gx, axis=-1, keepdims=True)
          + jnp.sum(t, axis=-1, keepdims=True)) / pool_w
    var = (jnp.sum((gm - mu) ** 2, axis=-1, keepdims=True)
           + jnp.sum((gx - mu) ** 2, axis=-1, keepdims=True)
           + jnp.sum((t - mu) ** 2, axis=-1, keepdims=True)) / pool_w
    sd = jnp.sqrt(var + 1e-5)
    z1 = (gm - mu) / sd * g1_ref[...] + b1_ref[...]
    z2 = (gx - mu) / sd * g2_ref[...] + b2_ref[...]
    z3 = (t - mu) / sd * g3_ref[...] + b3_ref[...]
    z = (jnp.dot(z1, W1a_ref[...], preferred_element_type=jnp.float32)
         + jnp.dot(z2, W1b_ref[...], preferred_element_type=jnp.float32)
         + jnp.dot(z3, W1c_ref[...], preferred_element_type=jnp.float32)
         + cb1_ref[...])
    z = jnp.maximum(z, 0.0)
    o_ref[...] = (jnp.dot(z, cW2_ref[...], preferred_element_type=jnp.float32)
                  + cb2_ref[...])


def _pool_classify(acc, degp, h, wl, bl, wr, g, b,
                   bt, tf8, tsW1, tsb1, tsg, tsb, tsW2, tsb2,
                   g1, g2, g3, b1, b2, b3, W1a, W1b, W1c, cb1, cW2, cb2):
  def full(shape):
    return pl.BlockSpec(shape, lambda *_: tuple(0 for _ in shape))
  return pl.pallas_call(
      _pool_body,
      grid=(GRID,),
      in_specs=[
          pl.BlockSpec((NC, RB, H), lambda i: (0, i, 0)),
          pl.BlockSpec((NC, RB, H), lambda i: (0, i, 0)),
          pl.BlockSpec((RB, H), lambda i: (i, 0)),
          full((H, H)), full((1, H)), full((H, H)), full((1, H)),
          full((1, H)),
          pl.BlockSpec((RB, 1), lambda i: (i, 0)),
          full((B, 8)), full((8, H // 2)), full((1, H // 2)),
          full((1, H // 2)), full((1, H // 2)), full((H // 2, H // 2)),
          full((1, H // 2)),
          full((1, H)), full((1, H)), full((1, H // 2)),
          full((1, H)), full((1, H)), full((1, H // 2)),
          full((H, H)), full((H, H)), full((H // 2, H)),
          full((1, H)), full((H, NUM_CLASSES)), full((1, NUM_CLASSES)),
      ],
      out_specs=pl.BlockSpec((B, NUM_CLASSES), lambda i: (0, 0)),
      out_shape=jax.ShapeDtypeStruct((B, NUM_CLASSES), jnp.float32),
      scratch_shapes=[
          pltpu.VMEM((B, H), jnp.float32),
          pltpu.VMEM((B, H), jnp.float32),
          pltpu.VMEM((B, H), jnp.float32),
      ],
  )(acc, degp, h, wl, bl, wr, g, b, bt, tf8, tsW1, tsb1, tsg, tsb, tsW2,
    tsb2, g1, g2, g3, b1, b2, b3, W1a, W1b, W1c, cb1, cW2, cb2)


def kernel(x, edge_index, batch, trackster_features, enc_W, enc_b, conv_Wl,
           conv_bl, conv_Wr, norm_g, norm_b, ts_W1, ts_b1, ts_ln_g, ts_ln_b,
           ts_W2, ts_b2, cls_ln_g, cls_ln_b, cls_W1, cls_b1, cls_W2, cls_b2):
  f32 = jnp.float32

  x8 = jnp.zeros((NP, 8), f32).at[:N, :F_IN].set(x.astype(f32))
  w8 = jnp.zeros((8, H), f32).at[:F_IN].set(enc_W.astype(f32))
  src2d = edge_index[0].astype(jnp.int32).reshape(NW, NB_I, CH_B, CHUNK)
  dst2d = edge_index[1].astype(jnp.int32).reshape(NW, NB_I, CH_B, CHUNK)
  bt = jnp.full((NP, 1), B, jnp.int32).at[:N, 0].set(batch.astype(jnp.int32))
  tf8 = jnp.zeros((B, 8), f32).at[:, :3].set(trackster_features.astype(f32))
  tsW1_8 = jnp.zeros((8, H // 2), f32).at[:3].set(ts_W1.astype(f32))

  g1 = cls_ln_g[None, :H]
  g2 = cls_ln_g[None, H:2 * H]
  g3 = cls_ln_g[None, 2 * H:]
  b1 = cls_ln_b[None, :H]
  b2 = cls_ln_b[None, H:2 * H]
  b3 = cls_ln_b[None, 2 * H:]
  W1a = cls_W1[:H]
  W1b = cls_W1[H:2 * H]
  W1c = cls_W1[2 * H:]

  h = _encoder(x8, w8, enc_b[None])

  degp = _deg_count(dst2d)
  for i in range(2):
    acc = _seg_sum(h, src2d, dst2d)
    h = _update(acc, degp, h, conv_Wl[i], conv_bl[i][None], conv_Wr[i],
                norm_g[i][None], norm_b[i][None])
  acc = _seg_sum(h, src2d, dst2d)

  return _pool_classify(
      acc, degp, h, conv_Wl[2], conv_bl[2][None], conv_Wr[2],
      norm_g[2][None], norm_b[2][None],
      bt, tf8, tsW1_8, ts_b1[None], ts_ln_g[None], ts_ln_b[None],
      ts_W2, ts_b2[None], g1, g2, g3, b1, b2, b3, W1a, W1b, W1c,
      cls_b1[None], cls_W2, cls_b2[None])
